# baseline (speedup 1.0000x reference)
import functools

import jax
import jax.numpy as jnp
from jax import lax
from jax.experimental import pallas as pl
from jax.experimental.pallas import tpu as pltpu
from jax.experimental.pallas import tpu_sc as plsc

_pcall = pl.pallas_call

K_ATTN = 16
K_DOWN = 16
K_UP = 3
K_UP_PAD = 8

_SC_NC = 2
_SC_NS = 16
_SC_NW = _SC_NC * _SC_NS


def _knn_body(k, nr, q_ref, rt_ref, o_ref):
    b = pl.program_id(0)
    q = q_ref[0]
    rt = rt_ref[0]
    qq = jnp.sum(q * q, axis=1, keepdims=True)
    rr = jnp.sum(rt * rt, axis=0, keepdims=True)
    cross = jnp.dot(q, rt)
    d = (qq - 2.0 * cross) + rr
    iota = lax.broadcasted_iota(jnp.int32, d.shape, 1)
    cols = []
    for _ in range(k):
        am = jnp.argmin(d, axis=1).astype(jnp.int32)[:, None]
        cols.append(am)
        d = jnp.where(iota == am, jnp.float32(jnp.inf), d)
    idx = jnp.concatenate(cols, axis=1)
    o_ref[0] = idx + b * nr


def _knn(qc, rc, k):
    B, nq, _ = qc.shape
    nr = rc.shape[1]
    rt = jnp.swapaxes(rc, 1, 2)
    bq = min(nq, 256)
    return _pcall(
        functools.partial(_knn_body, k, nr),
        grid=(B, nq // bq),
        in_specs=[pl.BlockSpec((1, bq, 3), lambda b, i: (b, i, 0)),
                  pl.BlockSpec((1, 3, nr), lambda b, i: (b, 0, 0))],
        out_specs=pl.BlockSpec((1, bq, k), lambda b, i: (b, i, 0)),
        out_shape=jax.ShapeDtypeStruct((B, nq, k), jnp.int32),
        cost_estimate=pl.CostEstimate(
            flops=B * nq * nr * (4 * k + 12), transcendentals=0,
            bytes_accessed=B * (nq + nr) * 12 + B * nq * k * 4),
    )(qc, rt)


def _gather_chunk(bpw, D):
    for c in range(128, 0, -8):
        if bpw % c == 0 and c * D * 4 <= 200_000 and (bpw // c) % 2 != 1:
            return c
    if bpw * D * 4 <= 200_000:
        return bpw
    raise ValueError((bpw, D))


def _gather_rows(table, idx):
    T, D = table.shape
    (M,) = idx.shape
    bpw = M // _SC_NW
    chunk = _gather_chunk(bpw, D)
    nchunks = bpw // chunk
    mesh = plsc.VectorSubcoreMesh(core_axis_name="c", subcore_axis_name="s")

    @functools.partial(
        pl.kernel, mesh=mesh,
        out_type=jax.ShapeDtypeStruct((M, D), jnp.float32),
        scratch_types=[pltpu.VMEM((bpw,), jnp.int32),
                       pltpu.VMEM((chunk, D), jnp.float32),
                       pltpu.VMEM((chunk, D), jnp.float32),
                       pltpu.SemaphoreType.DMA,
                       pltpu.SemaphoreType.DMA,
                       pltpu.SemaphoreType.DMA,
                       pltpu.SemaphoreType.DMA],
        compiler_params=pltpu.CompilerParams(use_tc_tiling_on_sc=False),
        cost_estimate=pl.CostEstimate(
            flops=0, transcendentals=0,
            bytes_accessed=2 * M * D * 4 + M * 4),
    )
    def kfn(table_hbm, idx_hbm, out_hbm, idx_v, buf0, buf1,
            gsem0, gsem1, ssem0, ssem1):
        wid = lax.axis_index("s") * _SC_NC + lax.axis_index("c")
        base = wid * bpw
        pltpu.sync_copy(idx_hbm.at[pl.ds(base, bpw)], idx_v)

        if nchunks == 1:
            pltpu.async_copy(table_hbm.at[idx_v], buf0, gsem0).wait()
            pltpu.sync_copy(buf0, out_hbm.at[pl.ds(base, bpw)])
            return

        def body(i, carry):
            c0 = 2 * i * chunk
            c1 = c0 + chunk
            g0 = pltpu.async_copy(
                table_hbm.at[idx_v.at[pl.ds(c0, chunk)]], buf0, gsem0)
            g1 = pltpu.async_copy(
                table_hbm.at[idx_v.at[pl.ds(c1, chunk)]], buf1, gsem1)
            g0.wait()
            s0 = pltpu.async_copy(buf0, out_hbm.at[pl.ds(base + c0, chunk)],
                                  ssem0)
            g1.wait()
            s1 = pltpu.async_copy(buf1, out_hbm.at[pl.ds(base + c1, chunk)],
                                  ssem1)
            s0.wait()
            s1.wait()
            return carry

        lax.fori_loop(0, nchunks // 2, body, 0)

    return kfn(table, idx)


def _rowcall(body, rows, row_mults, auxs, out_mult, out_dim, r_total, br,
             out_dtype=jnp.float32):
    grid = (r_total // br,)
    in_specs = []
    for a, m in zip(rows, row_mults):
        in_specs.append(pl.BlockSpec((br * m, a.shape[1]), lambda i: (i, 0)))
    for a in auxs:
        in_specs.append(pl.BlockSpec(a.shape, lambda i: (0,) * a.ndim))
    return _pcall(
        body, grid=grid, in_specs=in_specs,
        out_specs=pl.BlockSpec((br * out_mult, out_dim), lambda i: (i, 0)),
        out_shape=jax.ShapeDtypeStruct((r_total * out_mult, out_dim),
                                       out_dtype),
    )(*rows, *auxs)


def _dot(a, b):
    return jnp.dot(a, b, preferred_element_type=jnp.float32)


def _qkv_body(x_ref, win, bin_, wq, bq_, wkv, bkv, q_ref, kv_ref):
    x = x_ref[...]
    h = _dot(x, win[...]) + bin_[...]
    q_ref[...] = _dot(h, wq[...]) + bq_[...]
    kv_ref[...] = _dot(h, wkv[...]) + bkv[...]


def _attn_body(dm, g_ref, q_ref, cq_ref, f_ref,
               p1, p1b, p2, p2b, a1, a1b, a2, a2b, lo, lob, o_ref):
    K = K_ATTN
    br = q_ref.shape[0]
    g = g_ref[...]
    knb = g[:, :dm]
    vnb = g[:, dm:2 * dm]
    pnb = g[:, 2 * dm:2 * dm + 3]

    def rep(x):
        return jnp.broadcast_to(
            x[:, None, :], (br, K, x.shape[1])).reshape(br * K, x.shape[1])

    rel = rep(cq_ref[...]) - pnb
    pos = jnp.maximum(_dot(rel, p1[...]) + p1b[...], 0.0)
    pos = _dot(pos, p2[...]) + p2b[...]
    a = rep(q_ref[...]) - knb + pos
    a = jnp.maximum(_dot(a, a1[...]) + a1b[...], 0.0)
    a = _dot(a, a2[...]) + a2b[...]
    a3 = a.reshape(br, K, dm)
    amax = jnp.max(a3, axis=1, keepdims=True)
    e = jnp.exp(a3 - amax)
    w = e / jnp.sum(e, axis=1, keepdims=True)
    v3 = (vnb + pos).reshape(br, K, dm)
    out = jnp.sum(w * v3, axis=1)
    o_ref[...] = f_ref[...] + _dot(out, lo[...]) + lob[...]


def _down_body(g_ref, w, b, o_ref):
    h = jnp.maximum(_dot(g_ref[...], w[...]) + b[...], 0.0)
    br, dout = o_ref.shape
    o_ref[...] = jnp.max(h.reshape(br, K_DOWN, dout), axis=1)


def _up_body(dlow, g_ref, cq_ref, fh_ref, ws, bs, o_ref):
    K = K_UP_PAD
    br = cq_ref.shape[0]
    g = g_ref[...]
    nb = g[:, :dlow]
    pnb = g[:, dlow:dlow + 3]
    cqn = jnp.broadcast_to(
        cq_ref[...][:, None, :], (br, K, 3)).reshape(br * K, 3)
    diff = cqn - pnb
    dist = jnp.sum(diff * diff, axis=1, keepdims=True)
    w = 1.0 / (dist + 1e-8)
    w3 = w.reshape(br, K, 1)
    kio = lax.broadcasted_iota(jnp.int32, (br, K, 1), 1)
    w3 = jnp.where(kio < K_UP, w3, 0.0)
    w3 = w3 / jnp.sum(w3, axis=1, keepdims=True)
    interp = jnp.sum(w3 * nb.reshape(br, K, dlow), axis=1)
    o_ref[...] = interp + _dot(fh_ref[...], ws[...]) + bs[...]


def _mlp_body(logsm, x_ref, w1, b1, w2, b2, o_ref):
    h = jnp.maximum(_dot(x_ref[...], w1[...]) + b1[...], 0.0)
    y = _dot(h, w2[...]) + b2[...]
    if logsm:
        m = jnp.max(y, axis=1, keepdims=True)
        e = y - m
        y = e - jnp.log(jnp.sum(jnp.exp(e), axis=1, keepdims=True))
    o_ref[...] = y


def _lin_body(x_ref, w, b, o_ref):
    o_ref[...] = _dot(x_ref[...], w[...]) + b[...]


def _b2(b):
    return b.reshape(1, -1)


def _mlp2(x, mp, logsm=False):
    (w1, b1), (w2, b2) = mp
    r = x.shape[0]
    br = min(r, 1024)
    return _rowcall(functools.partial(_mlp_body, logsm), [x], [1],
                    [w1, _b2(b1), w2, _b2(b2)], 1, w2.shape[1], r, br)


def _lin(x, wb):
    w, b = wb
    r = x.shape[0]
    br = min(r, 1024)
    return _rowcall(_lin_body, [x], [1], [w, _b2(b)], 1, w.shape[1], r, br)


def _pt_block(cb, f, p, idx):
    B, N, _ = cb.shape
    d = f.shape[1]
    R = B * N
    (win, bin_), (wq, bq_) = p['lin_in'], p['q']
    (wk, bk), (wv, bv) = p['k'], p['v']
    wkv = jnp.concatenate([wk, wv], axis=1)
    bkv = jnp.concatenate([bk, bv])
    br = min(R, 512)
    q, kv = _pcall(
        _qkv_body, grid=(R // br,),
        in_specs=[pl.BlockSpec((br, f.shape[1]), lambda i: (i, 0))]
        + [pl.BlockSpec(a.shape, lambda i: (0, 0))
           for a in (win, _b2(bin_), wq, _b2(bq_), wkv, _b2(bkv))],
        out_specs=[pl.BlockSpec((br, d), lambda i: (i, 0)),
                   pl.BlockSpec((br, 2 * d), lambda i: (i, 0))],
        out_shape=[jax.ShapeDtypeStruct((R, d), jnp.float32),
                   jax.ShapeDtypeStruct((R, 2 * d), jnp.float32)],
    )(f, win, _b2(bin_), wq, _b2(bq_), wkv, _b2(bkv))
    cflat = cb.reshape(R, 3)
    table = jnp.concatenate(
        [kv, jnp.pad(cflat, ((0, 0), (0, 13)))], axis=1)
    g = _gather_rows(table, idx.reshape(-1))
    (p1, p1b), (p2, p2b) = p['pos1'], p['pos2']
    (a1, a1b), (a2, a2b) = p['attn1'], p['attn2']
    (lo, lob) = p['lin_out']
    br = min(R, 512)
    return _rowcall(
        functools.partial(_attn_body, d), [g, q, cflat, f], [K_ATTN, 1, 1, 1],
        [p1, _b2(p1b), p2, _b2(p2b), a1, _b2(a1b), a2, _b2(a2b),
         lo, _b2(lob)], 1, d, R, br)


def _tdown(cb, f, wb, n_out, idx):
    B = cb.shape[0]
    nc = cb[:, :n_out]
    g = _gather_rows(f, idx.reshape(-1))
    w, b = wb
    r = B * n_out
    br = min(r, 512)
    out = _rowcall(_down_body, [g], [K_DOWN], [w, _b2(b)],
                   1, w.shape[1], r, br)
    return nc, out


def _tup(cl, f_low, ch, f_high, p, idx3):
    B, nl, _ = cl.shape
    nh = ch.shape[1]
    fl = _lin(f_low, p['low'])
    d = fl.shape[1]
    idx8 = jnp.concatenate(
        [idx3, jnp.broadcast_to(idx3[..., :1], (B, nh, K_UP_PAD - K_UP))],
        axis=2)
    table = jnp.concatenate(
        [fl, jnp.pad(cl.reshape(B * nl, 3), ((0, 0), (0, 13)))], axis=1)
    g = _gather_rows(table, idx8.reshape(-1))
    ws, bs = p['skip']
    r = B * nh
    br = min(r, 512)
    return _rowcall(functools.partial(_up_body, d),
                    [g, ch.reshape(r, 3), f_high], [K_UP_PAD, 1, 1],
                    [ws, _b2(bs)], 1, d, r, br)


def kernel(coords, features, params):
    B, N, _ = coords.shape
    c0 = coords
    c1, c2, c3, c4 = (c0[:, :N // 4], c0[:, :N // 16],
                      c0[:, :N // 64], c0[:, :N // 256])
    n4 = N // 256
    base = jnp.arange(n4, dtype=jnp.int32)[None, None, :]
    off = jnp.arange(B, dtype=jnp.int32)[:, None, None] * n4
    idx_a4 = jnp.broadcast_to(base + off, (B, n4, n4))
    idx_a0 = _knn(c0, c0, K_ATTN)
    idx_a1 = _knn(c1, c1, K_ATTN)
    idx_a2 = _knn(c2, c2, K_ATTN)
    idx_a3 = _knn(c3, c3, K_ATTN)
    idx_d1 = _knn(c1, c0, K_DOWN)
    idx_d2 = _knn(c2, c1, K_DOWN)
    idx_d3 = _knn(c3, c2, K_DOWN)
    idx_d4 = _knn(c4, c3, K_DOWN)
    idx_u6 = _knn(c3, c4, K_UP)
    idx_u7 = _knn(c2, c3, K_UP)
    idx_u8 = _knn(c1, c2, K_UP)
    idx_u9 = _knn(c0, c1, K_UP)

    f = features.reshape(B * N, 3)
    f0 = _mlp2(f, params['mlp0'])
    f0 = _pt_block(c0, f0, params['pt0'], idx_a0)
    _, f1 = _tdown(c0, f0, params['td1'], N // 4, idx_d1)
    f1 = _pt_block(c1, f1, params['pt1'], idx_a1)
    _, f2 = _tdown(c1, f1, params['td2'], N // 16, idx_d2)
    f2 = _pt_block(c2, f2, params['pt2'], idx_a2)
    _, f3 = _tdown(c2, f2, params['td3'], N // 64, idx_d3)
    f3 = _pt_block(c3, f3, params['pt3'], idx_a3)
    _, f4 = _tdown(c3, f3, params['td4'], N // 256, idx_d4)
    f4 = _pt_block(c4, f4, params['pt4'], idx_a4)
    f5 = _mlp2(f4, params['mlp2'])
    f6 = _pt_block(c4, f5, params['pt5'], idx_a4)
    f7 = _tup(c4, f6, c3, f3, params['tu6'], idx_u6)
    f7 = _pt_block(c3, f7, params['pt6'], idx_a3)
    f8 = _tup(c3, f7, c2, f2, params['tu7'], idx_u7)
    f8 = _pt_block(c2, f8, params['pt7'], idx_a2)
    f9 = _tup(c2, f8, c1, f1, params['tu8'], idx_u8)
    f9 = _pt_block(c1, f9, params['pt8'], idx_a1)
    f10 = _tup(c1, f9, c0, f0, params['tu9'], idx_u9)
    f10 = _pt_block(c0, f10, params['pt9'], idx_a0)
    logits = _mlp2(f10, params['mlp3'], logsm=True)
    return logits.reshape(B, N, -1)


# speedup vs baseline: 1.1664x; 1.1664x over previous
import functools

import jax
import jax.numpy as jnp
from jax import lax
from jax.experimental import pallas as pl
from jax.experimental.pallas import tpu as pltpu
from jax.experimental.pallas import tpu_sc as plsc

_pcall = pl.pallas_call

K_ATTN = 16
K_DOWN = 16
K_UP = 3
K_UP_PAD = 8

_SC_NC = 2
_SC_NS = 16
_SC_NW = _SC_NC * _SC_NS


def _knn_body(k, nr, q_ref, rt_ref, o_ref):
    b = pl.program_id(0)
    q = q_ref[0]
    rt = rt_ref[0]
    qq = jnp.sum(q * q, axis=1, keepdims=True)
    rr = jnp.sum(rt * rt, axis=0, keepdims=True)
    cross = jnp.dot(q, rt)
    d = (qq - 2.0 * cross) + rr
    iota = lax.broadcasted_iota(jnp.int32, d.shape, 1)
    cols = []
    for _ in range(k):
        am = jnp.argmin(d, axis=1).astype(jnp.int32)[:, None]
        cols.append(am)
        d = jnp.where(iota == am, jnp.float32(jnp.inf), d)
    idx = jnp.concatenate(cols, axis=1)
    o_ref[0] = idx + b * nr


def _knn(qc, rc, k):
    B, nq, _ = qc.shape
    nr = rc.shape[1]
    rt = jnp.swapaxes(rc, 1, 2)
    bq = min(nq, 256)
    return _pcall(
        functools.partial(_knn_body, k, nr),
        grid=(B, nq // bq),
        in_specs=[pl.BlockSpec((1, bq, 3), lambda b, i: (b, i, 0)),
                  pl.BlockSpec((1, 3, nr), lambda b, i: (b, 0, 0))],
        out_specs=pl.BlockSpec((1, bq, k), lambda b, i: (b, i, 0)),
        out_shape=jax.ShapeDtypeStruct((B, nq, k), jnp.int32),
    )(qc, rt)


def _gather_chunk(bpw, D):
    for c in range(128, 0, -8):
        if bpw % c == 0 and c * D * 4 <= 200_000 and (bpw // c) % 2 != 1:
            return c
    if bpw * D * 4 <= 200_000:
        return bpw
    raise ValueError((bpw, D))


def _gather_rows(table, idx):
    T, D = table.shape
    (M,) = idx.shape
    tiled = M % 4096 == 0
    if tiled and D % 128 != 0:
        Dp = (D + 127) // 128 * 128
        table = jnp.pad(table, ((0, 0), (0, Dp - D)))
        D = Dp
    bpw = M // _SC_NW
    chunk = _gather_chunk(bpw, D)
    nchunks = bpw // chunk
    mesh = plsc.VectorSubcoreMesh(core_axis_name="c", subcore_axis_name="s")

    @functools.partial(
        pl.kernel, mesh=mesh,
        out_type=jax.ShapeDtypeStruct((M, D), jnp.float32),
        scratch_types=[pltpu.VMEM((bpw,), jnp.int32),
                       pltpu.VMEM((chunk, D), jnp.float32),
                       pltpu.VMEM((chunk, D), jnp.float32),
                       pltpu.SemaphoreType.DMA,
                       pltpu.SemaphoreType.DMA,
                       pltpu.SemaphoreType.DMA,
                       pltpu.SemaphoreType.DMA],
        compiler_params=pltpu.CompilerParams(use_tc_tiling_on_sc=tiled),
    )
    def kfn(table_hbm, idx_hbm, out_hbm, idx_v, buf0, buf1,
            gsem0, gsem1, ssem0, ssem1):
        wid = lax.axis_index("s") * _SC_NC + lax.axis_index("c")
        base = wid * bpw
        pltpu.sync_copy(idx_hbm.at[pl.ds(base, bpw)], idx_v)

        if nchunks == 1:
            pltpu.async_copy(table_hbm.at[idx_v], buf0, gsem0).wait()
            pltpu.sync_copy(buf0, out_hbm.at[pl.ds(base, bpw)])
            return

        def body(i, carry):
            c0 = 2 * i * chunk
            c1 = c0 + chunk
            g0 = pltpu.async_copy(
                table_hbm.at[idx_v.at[pl.ds(c0, chunk)]], buf0, gsem0)
            g1 = pltpu.async_copy(
                table_hbm.at[idx_v.at[pl.ds(c1, chunk)]], buf1, gsem1)
            g0.wait()
            s0 = pltpu.async_copy(buf0, out_hbm.at[pl.ds(base + c0, chunk)],
                                  ssem0)
            g1.wait()
            s1 = pltpu.async_copy(buf1, out_hbm.at[pl.ds(base + c1, chunk)],
                                  ssem1)
            s0.wait()
            s1.wait()
            return carry

        lax.fori_loop(0, nchunks // 2, body, 0)

    return kfn(table, idx)


def _rowcall(body, rows, row_mults, auxs, out_mult, out_dim, r_total, br,
             out_dtype=jnp.float32):
    grid = (r_total // br,)
    in_specs = []
    for a, m in zip(rows, row_mults):
        in_specs.append(pl.BlockSpec((br * m, a.shape[1]), lambda i: (i, 0)))
    for a in auxs:
        in_specs.append(pl.BlockSpec(a.shape, lambda i: (0,) * a.ndim))
    return _pcall(
        body, grid=grid, in_specs=in_specs,
        out_specs=pl.BlockSpec((br * out_mult, out_dim), lambda i: (i, 0)),
        out_shape=jax.ShapeDtypeStruct((r_total * out_mult, out_dim),
                                       out_dtype),
    )(*rows, *auxs)


def _dot(a, b):
    return jnp.dot(a, b, preferred_element_type=jnp.float32)


def _qkv_body(x_ref, win, bin_, wq, bq_, wkv, bkv, q_ref, kv_ref):
    x = x_ref[...]
    h = _dot(x, win[...]) + bin_[...]
    q_ref[...] = _dot(h, wq[...]) + bq_[...]
    kv_ref[...] = _dot(h, wkv[...]) + bkv[...]


def _attn_body(dm, g_ref, q_ref, cq_ref, f_ref,
               p1, p1b, p2, p2b, a1, a1b, a2, a2b, lo, lob, o_ref):
    K = K_ATTN
    br = q_ref.shape[0]
    g = g_ref[...]
    knb = g[:, :dm]
    vnb = g[:, dm:2 * dm]
    pnb = g[:, 2 * dm:2 * dm + 3]

    def rep(x):
        return jnp.broadcast_to(
            x[:, None, :], (br, K, x.shape[1])).reshape(br * K, x.shape[1])

    rel = rep(cq_ref[...]) - pnb
    pos = jnp.maximum(_dot(rel, p1[...]) + p1b[...], 0.0)
    pos = _dot(pos, p2[...]) + p2b[...]
    a = rep(q_ref[...]) - knb + pos
    a = jnp.maximum(_dot(a, a1[...]) + a1b[...], 0.0)
    a = _dot(a, a2[...]) + a2b[...]
    a3 = a.reshape(br, K, dm)
    amax = jnp.max(a3, axis=1, keepdims=True)
    e = jnp.exp(a3 - amax)
    w = e / jnp.sum(e, axis=1, keepdims=True)
    v3 = (vnb + pos).reshape(br, K, dm)
    out = jnp.sum(w * v3, axis=1)
    o_ref[...] = f_ref[...] + _dot(out, lo[...]) + lob[...]


def _down_body(g_ref, w, b, o_ref):
    din = w.shape[0]
    h = jnp.maximum(_dot(g_ref[...][:, :din], w[...]) + b[...], 0.0)
    br, dout = o_ref.shape
    o_ref[...] = jnp.max(h.reshape(br, K_DOWN, dout), axis=1)


def _up_body(dlow, g_ref, cq_ref, fh_ref, ws, bs, o_ref):
    K = K_UP_PAD
    br = cq_ref.shape[0]
    g = g_ref[...]
    nb = g[:, :dlow]
    pnb = g[:, dlow:dlow + 3]
    cqn = jnp.broadcast_to(
        cq_ref[...][:, None, :], (br, K, 3)).reshape(br * K, 3)
    diff = cqn - pnb
    dist = jnp.sum(diff * diff, axis=1, keepdims=True)
    w = 1.0 / (dist + 1e-8)
    w3 = w.reshape(br, K, 1)
    kio = lax.broadcasted_iota(jnp.int32, (br, K, 1), 1)
    w3 = jnp.where(kio < K_UP, w3, 0.0)
    w3 = w3 / jnp.sum(w3, axis=1, keepdims=True)
    interp = jnp.sum(w3 * nb.reshape(br, K, dlow), axis=1)
    o_ref[...] = interp + _dot(fh_ref[...], ws[...]) + bs[...]


def _mlp_body(logsm, x_ref, w1, b1, w2, b2, o_ref):
    h = jnp.maximum(_dot(x_ref[...], w1[...]) + b1[...], 0.0)
    y = _dot(h, w2[...]) + b2[...]
    if logsm:
        m = jnp.max(y, axis=1, keepdims=True)
        e = y - m
        y = e - jnp.log(jnp.sum(jnp.exp(e), axis=1, keepdims=True))
    o_ref[...] = y


def _lin_body(x_ref, w, b, o_ref):
    o_ref[...] = _dot(x_ref[...], w[...]) + b[...]


def _b2(b):
    return b.reshape(1, -1)


def _mlp2(x, mp, logsm=False):
    (w1, b1), (w2, b2) = mp
    r = x.shape[0]
    br = min(r, 1024)
    return _rowcall(functools.partial(_mlp_body, logsm), [x], [1],
                    [w1, _b2(b1), w2, _b2(b2)], 1, w2.shape[1], r, br)


def _lin(x, wb):
    w, b = wb
    r = x.shape[0]
    br = min(r, 1024)
    return _rowcall(_lin_body, [x], [1], [w, _b2(b)], 1, w.shape[1], r, br)


def _pt_block(cb, f, p, idx):
    B, N, _ = cb.shape
    d = f.shape[1]
    R = B * N
    (win, bin_), (wq, bq_) = p['lin_in'], p['q']
    (wk, bk), (wv, bv) = p['k'], p['v']
    wkv = jnp.concatenate([wk, wv], axis=1)
    bkv = jnp.concatenate([bk, bv])
    br = min(R, 512)
    q, kv = _pcall(
        _qkv_body, grid=(R // br,),
        in_specs=[pl.BlockSpec((br, f.shape[1]), lambda i: (i, 0))]
        + [pl.BlockSpec(a.shape, lambda i: (0, 0))
           for a in (win, _b2(bin_), wq, _b2(bq_), wkv, _b2(bkv))],
        out_specs=[pl.BlockSpec((br, d), lambda i: (i, 0)),
                   pl.BlockSpec((br, 2 * d), lambda i: (i, 0))],
        out_shape=[jax.ShapeDtypeStruct((R, d), jnp.float32),
                   jax.ShapeDtypeStruct((R, 2 * d), jnp.float32)],
    )(f, win, _b2(bin_), wq, _b2(bq_), wkv, _b2(bkv))
    cflat = cb.reshape(R, 3)
    table = jnp.concatenate(
        [kv, jnp.pad(cflat, ((0, 0), (0, 13)))], axis=1)
    g = _gather_rows(table, idx.reshape(-1))
    (p1, p1b), (p2, p2b) = p['pos1'], p['pos2']
    (a1, a1b), (a2, a2b) = p['attn1'], p['attn2']
    (lo, lob) = p['lin_out']
    br = min(R, 512)
    return _rowcall(
        functools.partial(_attn_body, d), [g, q, cflat, f], [K_ATTN, 1, 1, 1],
        [p1, _b2(p1b), p2, _b2(p2b), a1, _b2(a1b), a2, _b2(a2b),
         lo, _b2(lob)], 1, d, R, br)


def _tdown(cb, f, wb, n_out, idx):
    B = cb.shape[0]
    nc = cb[:, :n_out]
    g = _gather_rows(f, idx.reshape(-1))
    w, b = wb
    r = B * n_out
    br = min(r, 512)
    out = _rowcall(_down_body, [g], [K_DOWN], [w, _b2(b)],
                   1, w.shape[1], r, br)
    return nc, out


def _tup(cl, f_low, ch, f_high, p, idx3):
    B, nl, _ = cl.shape
    nh = ch.shape[1]
    fl = _lin(f_low, p['low'])
    d = fl.shape[1]
    idx8 = jnp.concatenate(
        [idx3, jnp.broadcast_to(idx3[..., :1], (B, nh, K_UP_PAD - K_UP))],
        axis=2)
    table = jnp.concatenate(
        [fl, jnp.pad(cl.reshape(B * nl, 3), ((0, 0), (0, 13)))], axis=1)
    g = _gather_rows(table, idx8.reshape(-1))
    ws, bs = p['skip']
    r = B * nh
    br = min(r, 512)
    return _rowcall(functools.partial(_up_body, d),
                    [g, ch.reshape(r, 3), f_high], [K_UP_PAD, 1, 1],
                    [ws, _b2(bs)], 1, d, r, br)


def kernel(coords, features, params):
    B, N, _ = coords.shape
    c0 = coords
    c1, c2, c3, c4 = (c0[:, :N // 4], c0[:, :N // 16],
                      c0[:, :N // 64], c0[:, :N // 256])
    n4 = N // 256
    base = jnp.arange(n4, dtype=jnp.int32)[None, None, :]
    off = jnp.arange(B, dtype=jnp.int32)[:, None, None] * n4
    idx_a4 = jnp.broadcast_to(base + off, (B, n4, n4))
    idx_a0 = _knn(c0, c0, K_ATTN)
    idx_a1 = _knn(c1, c1, K_ATTN)
    idx_a2 = _knn(c2, c2, K_ATTN)
    idx_a3 = _knn(c3, c3, K_ATTN)
    idx_d1 = _knn(c1, c0, K_DOWN)
    idx_d2 = _knn(c2, c1, K_DOWN)
    idx_d3 = _knn(c3, c2, K_DOWN)
    idx_d4 = _knn(c4, c3, K_DOWN)
    idx_u6 = _knn(c3, c4, K_UP)
    idx_u7 = _knn(c2, c3, K_UP)
    idx_u8 = _knn(c1, c2, K_UP)
    idx_u9 = _knn(c0, c1, K_UP)

    f = features.reshape(B * N, 3)
    f0 = _mlp2(f, params['mlp0'])
    f0 = _pt_block(c0, f0, params['pt0'], idx_a0)
    _, f1 = _tdown(c0, f0, params['td1'], N // 4, idx_d1)
    f1 = _pt_block(c1, f1, params['pt1'], idx_a1)
    _, f2 = _tdown(c1, f1, params['td2'], N // 16, idx_d2)
    f2 = _pt_block(c2, f2, params['pt2'], idx_a2)
    _, f3 = _tdown(c2, f2, params['td3'], N // 64, idx_d3)
    f3 = _pt_block(c3, f3, params['pt3'], idx_a3)
    _, f4 = _tdown(c3, f3, params['td4'], N // 256, idx_d4)
    f4 = _pt_block(c4, f4, params['pt4'], idx_a4)
    f5 = _mlp2(f4, params['mlp2'])
    f6 = _pt_block(c4, f5, params['pt5'], idx_a4)
    f7 = _tup(c4, f6, c3, f3, params['tu6'], idx_u6)
    f7 = _pt_block(c3, f7, params['pt6'], idx_a3)
    f8 = _tup(c3, f7, c2, f2, params['tu7'], idx_u7)
    f8 = _pt_block(c2, f8, params['pt7'], idx_a2)
    f9 = _tup(c2, f8, c1, f1, params['tu8'], idx_u8)
    f9 = _pt_block(c1, f9, params['pt8'], idx_a1)
    f10 = _tup(c1, f9, c0, f0, params['tu9'], idx_u9)
    f10 = _pt_block(c0, f10, params['pt9'], idx_a0)
    logits = _mlp2(f10, params['mlp3'], logsm=True)
    return logits.reshape(B, N, -1)


# speedup vs baseline: 1.1882x; 1.0187x over previous
import functools

import jax
import jax.numpy as jnp
from jax import lax
from jax.experimental import pallas as pl
from jax.experimental.pallas import tpu as pltpu
from jax.experimental.pallas import tpu_sc as plsc

_pcall = pl.pallas_call

K_ATTN = 16
K_DOWN = 16
K_UP = 3
K_UP_PAD = 8

_SC_NC = 2
_SC_NS = 16
_SC_NW = _SC_NC * _SC_NS


def _knn_body(k, nr, q_ref, rt_ref, o_ref):
    b = pl.program_id(0)
    q = q_ref[0]
    rt = rt_ref[0]
    qq = jnp.sum(q * q, axis=1, keepdims=True)
    rr = jnp.sum(rt * rt, axis=0, keepdims=True)
    cross = jnp.dot(q, rt)
    d = (qq - 2.0 * cross) + rr
    iota = lax.broadcasted_iota(jnp.int32, d.shape, 1)
    cols = []
    for _ in range(k):
        am = jnp.argmin(d, axis=1).astype(jnp.int32)[:, None]
        cols.append(am)
        d = jnp.where(iota == am, jnp.float32(jnp.inf), d)
    idx = jnp.concatenate(cols, axis=1)
    o_ref[0] = idx + b * nr


def _knn(qc, rc, k):
    B, nq, _ = qc.shape
    nr = rc.shape[1]
    rt = jnp.swapaxes(rc, 1, 2)
    bq = min(nq, 256)
    return _pcall(
        functools.partial(_knn_body, k, nr),
        grid=(B, nq // bq),
        in_specs=[pl.BlockSpec((1, bq, 3), lambda b, i: (b, i, 0)),
                  pl.BlockSpec((1, 3, nr), lambda b, i: (b, 0, 0))],
        out_specs=pl.BlockSpec((1, bq, k), lambda b, i: (b, i, 0)),
        out_shape=jax.ShapeDtypeStruct((B, nq, k), jnp.int32),
    )(qc, rt)


def _gather_chunk(bpw, D):
    for c in range(128, 0, -8):
        if bpw % c == 0 and c * D * 4 <= 200_000 and (bpw // c) % 2 != 1:
            return c
    if bpw * D * 4 <= 200_000:
        return bpw
    raise ValueError((bpw, D))


def _gather_rows(table, idx, force_untiled=False):
    T, D = table.shape
    (M,) = idx.shape
    tiled = M % 4096 == 0 and not force_untiled
    if tiled and D % 128 != 0:
        Dp = (D + 127) // 128 * 128
        table = jnp.pad(table, ((0, 0), (0, Dp - D)))
        D = Dp
    bpw = M // _SC_NW
    chunk = _gather_chunk(bpw, D)
    nchunks = bpw // chunk
    mesh = plsc.VectorSubcoreMesh(core_axis_name="c", subcore_axis_name="s")

    @functools.partial(
        pl.kernel, mesh=mesh,
        out_type=jax.ShapeDtypeStruct((M, D), jnp.float32),
        scratch_types=[pltpu.VMEM((bpw,), jnp.int32),
                       pltpu.VMEM((chunk, D), jnp.float32),
                       pltpu.VMEM((chunk, D), jnp.float32),
                       pltpu.SemaphoreType.DMA,
                       pltpu.SemaphoreType.DMA,
                       pltpu.SemaphoreType.DMA,
                       pltpu.SemaphoreType.DMA],
        compiler_params=pltpu.CompilerParams(use_tc_tiling_on_sc=tiled),
    )
    def kfn(table_hbm, idx_hbm, out_hbm, idx_v, buf0, buf1,
            gsem0, gsem1, ssem0, ssem1):
        wid = lax.axis_index("s") * _SC_NC + lax.axis_index("c")
        base = wid * bpw
        pltpu.sync_copy(idx_hbm.at[pl.ds(base, bpw)], idx_v)

        if nchunks == 1:
            pltpu.async_copy(table_hbm.at[idx_v], buf0, gsem0).wait()
            pltpu.sync_copy(buf0, out_hbm.at[pl.ds(base, bpw)])
            return

        def body(i, carry):
            c0 = 2 * i * chunk
            c1 = c0 + chunk
            g0 = pltpu.async_copy(
                table_hbm.at[idx_v.at[pl.ds(c0, chunk)]], buf0, gsem0)
            g1 = pltpu.async_copy(
                table_hbm.at[idx_v.at[pl.ds(c1, chunk)]], buf1, gsem1)
            g0.wait()
            s0 = pltpu.async_copy(buf0, out_hbm.at[pl.ds(base + c0, chunk)],
                                  ssem0)
            g1.wait()
            s1 = pltpu.async_copy(buf1, out_hbm.at[pl.ds(base + c1, chunk)],
                                  ssem1)
            s0.wait()
            s1.wait()
            return carry

        lax.fori_loop(0, nchunks // 2, body, 0)

    return kfn(table, idx)


def _rowcall(body, rows, row_mults, auxs, out_mult, out_dim, r_total, br,
             out_dtype=jnp.float32):
    grid = (r_total // br,)
    in_specs = []
    for a, m in zip(rows, row_mults):
        in_specs.append(pl.BlockSpec((br * m, a.shape[1]), lambda i: (i, 0)))
    for a in auxs:
        in_specs.append(pl.BlockSpec(a.shape, lambda i: (0,) * a.ndim))
    return _pcall(
        body, grid=grid, in_specs=in_specs,
        out_specs=pl.BlockSpec((br * out_mult, out_dim), lambda i: (i, 0)),
        out_shape=jax.ShapeDtypeStruct((r_total * out_mult, out_dim),
                                       out_dtype),
    )(*rows, *auxs)


def _dot(a, b):
    return jnp.dot(a, b, preferred_element_type=jnp.float32)


def _qkv_body(x_ref, win, bin_, wq, bq_, wkv, bkv, q_ref, kv_ref):
    x = x_ref[...]
    h = _dot(x, win[...]) + bin_[...]
    q_ref[...] = _dot(h, wq[...]) + bq_[...]
    kv_ref[...] = _dot(h, wkv[...]) + bkv[...]


def _attn_tail(dm, knb, vnb, pnb, q_ref, cq_ref, f_ref,
               p1, p1b, p2, p2b, a1, a1b, a2, a2b, lo, lob, o_ref):
    K = K_ATTN
    br = q_ref.shape[0]

    def rep(x):
        return jnp.broadcast_to(
            x[:, None, :], (br, K, x.shape[1])).reshape(br * K, x.shape[1])

    rel = rep(cq_ref[...]) - pnb
    pos = jnp.maximum(_dot(rel, p1[...]) + p1b[...], 0.0)
    pos = _dot(pos, p2[...]) + p2b[...]
    a = rep(q_ref[...]) - knb + pos
    a = jnp.maximum(_dot(a, a1[...]) + a1b[...], 0.0)
    a = _dot(a, a2[...]) + a2b[...]
    a3 = a.reshape(br, K, dm)
    amax = jnp.max(a3, axis=1, keepdims=True)
    e = jnp.exp(a3 - amax)
    w = e / jnp.sum(e, axis=1, keepdims=True)
    v3 = (vnb + pos).reshape(br, K, dm)
    out = jnp.sum(w * v3, axis=1)
    o_ref[...] = f_ref[...] + _dot(out, lo[...]) + lob[...]


def _attn_body(dm, g_ref, q_ref, cq_ref, f_ref,
               p1, p1b, p2, p2b, a1, a1b, a2, a2b, lo, lob, o_ref):
    g = g_ref[...]
    _attn_tail(dm, g[:, :dm], g[:, dm:2 * dm], g[:, 2 * dm:2 * dm + 3],
               q_ref, cq_ref, f_ref,
               p1, p1b, p2, p2b, a1, a1b, a2, a2b, lo, lob, o_ref)


def _attn_body_split(dm, g_ref, pc_ref, q_ref, cq_ref, f_ref,
                     p1, p1b, p2, p2b, a1, a1b, a2, a2b, lo, lob, o_ref):
    g = g_ref[...]
    _attn_tail(dm, g[:, :dm], g[:, dm:2 * dm], pc_ref[...][:, :3],
               q_ref, cq_ref, f_ref,
               p1, p1b, p2, p2b, a1, a1b, a2, a2b, lo, lob, o_ref)


def _attn_body_dense(dm, n, q_ref, cq_ref, f_ref, kv_ref, cb_ref,
                     p1, p1b, p2, p2b, a1, a1b, a2, a2b, lo, lob, o_ref):
    K = K_ATTN
    br = q_ref.shape[0]
    B = br // n
    kv = kv_ref[...].reshape(B, 1, n, 2 * dm)
    kv = jnp.broadcast_to(kv, (B, n, n, 2 * dm)).reshape(br * K, 2 * dm)
    cnb = cb_ref[...].reshape(B, 1, n, 3)
    cnb = jnp.broadcast_to(cnb, (B, n, n, 3)).reshape(br * K, 3)
    _attn_tail(dm, kv[:, :dm], kv[:, dm:2 * dm], cnb,
               q_ref, cq_ref, f_ref,
               p1, p1b, p2, p2b, a1, a1b, a2, a2b, lo, lob, o_ref)


def _down_body(g_ref, w, b, o_ref):
    din = w.shape[0]
    h = jnp.maximum(_dot(g_ref[...][:, :din], w[...]) + b[...], 0.0)
    br, dout = o_ref.shape
    o_ref[...] = jnp.max(h.reshape(br, K_DOWN, dout), axis=1)


def _up_body(dlow, g_ref, cq_ref, fh_ref, ws, bs, o_ref):
    K = K_UP_PAD
    br = cq_ref.shape[0]
    g = g_ref[...]
    nb = g[:, :dlow]
    pnb = g[:, dlow:dlow + 3]
    cqn = jnp.broadcast_to(
        cq_ref[...][:, None, :], (br, K, 3)).reshape(br * K, 3)
    diff = cqn - pnb
    dist = jnp.sum(diff * diff, axis=1, keepdims=True)
    w = 1.0 / (dist + 1e-8)
    w3 = w.reshape(br, K, 1)
    kio = lax.broadcasted_iota(jnp.int32, (br, K, 1), 1)
    w3 = jnp.where(kio < K_UP, w3, 0.0)
    w3 = w3 / jnp.sum(w3, axis=1, keepdims=True)
    interp = jnp.sum(w3 * nb.reshape(br, K, dlow), axis=1)
    o_ref[...] = interp + _dot(fh_ref[...], ws[...]) + bs[...]


def _mlp_body(logsm, x_ref, w1, b1, w2, b2, o_ref):
    h = jnp.maximum(_dot(x_ref[...], w1[...]) + b1[...], 0.0)
    y = _dot(h, w2[...]) + b2[...]
    if logsm:
        m = jnp.max(y, axis=1, keepdims=True)
        e = y - m
        y = e - jnp.log(jnp.sum(jnp.exp(e), axis=1, keepdims=True))
    o_ref[...] = y


def _lin_body(x_ref, w, b, o_ref):
    o_ref[...] = _dot(x_ref[...], w[...]) + b[...]


def _b2(b):
    return b.reshape(1, -1)


def _mlp2(x, mp, logsm=False):
    (w1, b1), (w2, b2) = mp
    r = x.shape[0]
    br = min(r, 1024)
    return _rowcall(functools.partial(_mlp_body, logsm), [x], [1],
                    [w1, _b2(b1), w2, _b2(b2)], 1, w2.shape[1], r, br)


def _lin(x, wb):
    w, b = wb
    r = x.shape[0]
    br = min(r, 1024)
    return _rowcall(_lin_body, [x], [1], [w, _b2(b)], 1, w.shape[1], r, br)


def _pt_block(cb, f, p, idx):
    B, N, _ = cb.shape
    d = f.shape[1]
    R = B * N
    (win, bin_), (wq, bq_) = p['lin_in'], p['q']
    (wk, bk), (wv, bv) = p['k'], p['v']
    wkv = jnp.concatenate([wk, wv], axis=1)
    bkv = jnp.concatenate([bk, bv])
    br = min(R, 512)
    q, kv = _pcall(
        _qkv_body, grid=(R // br,),
        in_specs=[pl.BlockSpec((br, f.shape[1]), lambda i: (i, 0))]
        + [pl.BlockSpec(a.shape, lambda i: (0, 0))
           for a in (win, _b2(bin_), wq, _b2(bq_), wkv, _b2(bkv))],
        out_specs=[pl.BlockSpec((br, d), lambda i: (i, 0)),
                   pl.BlockSpec((br, 2 * d), lambda i: (i, 0))],
        out_shape=[jax.ShapeDtypeStruct((R, d), jnp.float32),
                   jax.ShapeDtypeStruct((R, 2 * d), jnp.float32)],
    )(f, win, _b2(bin_), wq, _b2(bq_), wkv, _b2(bkv))
    cflat = cb.reshape(R, 3)
    (p1, p1b), (p2, p2b) = p['pos1'], p['pos2']
    (a1, a1b), (a2, a2b) = p['attn1'], p['attn2']
    (lo, lob) = p['lin_out']
    wts = [p1, _b2(p1b), p2, _b2(p2b), a1, _b2(a1b), a2, _b2(a2b),
           lo, _b2(lob)]
    br = min(R, 512)
    if N == K_ATTN:
        return _rowcall(
            functools.partial(_attn_body_dense, d, N),
            [q, cflat, f], [1, 1, 1], [kv, cflat] + wts, 1, d, R, br)
    if d == 32:
        table = jnp.concatenate(
            [kv, jnp.pad(cflat, ((0, 0), (0, 13)))], axis=1)
        g = _gather_rows(table, idx.reshape(-1))
        return _rowcall(
            functools.partial(_attn_body, d), [g, q, cflat, f],
            [K_ATTN, 1, 1, 1], wts, 1, d, R, br)
    g = _gather_rows(kv, idx.reshape(-1))
    pc = _gather_rows(jnp.pad(cflat, ((0, 0), (0, 13))), idx.reshape(-1),
                      force_untiled=True)
    return _rowcall(
        functools.partial(_attn_body_split, d), [g, pc, q, cflat, f],
        [K_ATTN, K_ATTN, 1, 1, 1], wts, 1, d, R, br)


def _tdown(cb, f, wb, n_out, idx):
    B = cb.shape[0]
    nc = cb[:, :n_out]
    g = _gather_rows(f, idx.reshape(-1))
    w, b = wb
    r = B * n_out
    br = min(r, 512)
    out = _rowcall(_down_body, [g], [K_DOWN], [w, _b2(b)],
                   1, w.shape[1], r, br)
    return nc, out


def _tup(cl, f_low, ch, f_high, p, idx3):
    B, nl, _ = cl.shape
    nh = ch.shape[1]
    fl = _lin(f_low, p['low'])
    d = fl.shape[1]
    idx8 = jnp.concatenate(
        [idx3, jnp.broadcast_to(idx3[..., :1], (B, nh, K_UP_PAD - K_UP))],
        axis=2)
    table = jnp.concatenate(
        [fl, jnp.pad(cl.reshape(B * nl, 3), ((0, 0), (0, 13)))], axis=1)
    g = _gather_rows(table, idx8.reshape(-1))
    ws, bs = p['skip']
    r = B * nh
    br = min(r, 512)
    return _rowcall(functools.partial(_up_body, d),
                    [g, ch.reshape(r, 3), f_high], [K_UP_PAD, 1, 1],
                    [ws, _b2(bs)], 1, d, r, br)


def kernel(coords, features, params):
    B, N, _ = coords.shape
    c0 = coords
    c1, c2, c3, c4 = (c0[:, :N // 4], c0[:, :N // 16],
                      c0[:, :N // 64], c0[:, :N // 256])
    n4 = N // 256
    base = jnp.arange(n4, dtype=jnp.int32)[None, None, :]
    off = jnp.arange(B, dtype=jnp.int32)[:, None, None] * n4
    idx_a4 = jnp.broadcast_to(base + off, (B, n4, n4))
    idx_a0 = _knn(c0, c0, K_ATTN)
    idx_a1 = _knn(c1, c1, K_ATTN)
    idx_a2 = _knn(c2, c2, K_ATTN)
    idx_a3 = _knn(c3, c3, K_ATTN)
    idx_d1 = _knn(c1, c0, K_DOWN)
    idx_d2 = _knn(c2, c1, K_DOWN)
    idx_d3 = _knn(c3, c2, K_DOWN)
    idx_d4 = _knn(c4, c3, K_DOWN)
    idx_u6 = _knn(c3, c4, K_UP)
    idx_u7 = _knn(c2, c3, K_UP)
    idx_u8 = _knn(c1, c2, K_UP)
    idx_u9 = _knn(c0, c1, K_UP)

    f = features.reshape(B * N, 3)
    f0 = _mlp2(f, params['mlp0'])
    f0 = _pt_block(c0, f0, params['pt0'], idx_a0)
    _, f1 = _tdown(c0, f0, params['td1'], N // 4, idx_d1)
    f1 = _pt_block(c1, f1, params['pt1'], idx_a1)
    _, f2 = _tdown(c1, f1, params['td2'], N // 16, idx_d2)
    f2 = _pt_block(c2, f2, params['pt2'], idx_a2)
    _, f3 = _tdown(c2, f2, params['td3'], N // 64, idx_d3)
    f3 = _pt_block(c3, f3, params['pt3'], idx_a3)
    _, f4 = _tdown(c3, f3, params['td4'], N // 256, idx_d4)
    f4 = _pt_block(c4, f4, params['pt4'], idx_a4)
    f5 = _mlp2(f4, params['mlp2'])
    f6 = _pt_block(c4, f5, params['pt5'], idx_a4)
    f7 = _tup(c4, f6, c3, f3, params['tu6'], idx_u6)
    f7 = _pt_block(c3, f7, params['pt6'], idx_a3)
    f8 = _tup(c3, f7, c2, f2, params['tu7'], idx_u7)
    f8 = _pt_block(c2, f8, params['pt7'], idx_a2)
    f9 = _tup(c2, f8, c1, f1, params['tu8'], idx_u8)
    f9 = _pt_block(c1, f9, params['pt8'], idx_a1)
    f10 = _tup(c1, f9, c0, f0, params['tu9'], idx_u9)
    f10 = _pt_block(c0, f10, params['pt9'], idx_a0)
    logits = _mlp2(f10, params['mlp3'], logsm=True)
    return logits.reshape(B, N, -1)


# speedup vs baseline: 1.2445x; 1.0474x over previous
import functools

import jax
import jax.numpy as jnp
from jax import lax
from jax.experimental import pallas as pl
from jax.experimental.pallas import tpu as pltpu
from jax.experimental.pallas import tpu_sc as plsc

_pcall = pl.pallas_call

K_ATTN = 16
K_DOWN = 16
K_UP = 3
K_UP_PAD = 8

_SC_NC = 2
_SC_NS = 16
_SC_NW = _SC_NC * _SC_NS


def _knn_body(k, nr, q_ref, rt_ref, o_ref):
    b = pl.program_id(0)
    q = q_ref[0]
    rt = rt_ref[0]
    qq = jnp.sum(q * q, axis=1, keepdims=True)
    rr = jnp.sum(rt * rt, axis=0, keepdims=True)
    cross = jnp.dot(q, rt)
    d = (qq - 2.0 * cross) + rr
    iota = lax.broadcasted_iota(jnp.int32, d.shape, 1)
    cols = []
    for _ in range(k):
        am = jnp.argmin(d, axis=1).astype(jnp.int32)[:, None]
        cols.append(am)
        d = jnp.where(iota == am, jnp.float32(jnp.inf), d)
    idx = jnp.concatenate(cols, axis=1)
    o_ref[0] = idx + b * nr


def _knn(qc, rc, k):
    B, nq, _ = qc.shape
    nr = rc.shape[1]
    rt = jnp.swapaxes(rc, 1, 2)
    bq = min(nq, 256)
    return _pcall(
        functools.partial(_knn_body, k, nr),
        grid=(B, nq // bq),
        in_specs=[pl.BlockSpec((1, bq, 3), lambda b, i: (b, i, 0)),
                  pl.BlockSpec((1, 3, nr), lambda b, i: (b, 0, 0))],
        out_specs=pl.BlockSpec((1, bq, k), lambda b, i: (b, i, 0)),
        out_shape=jax.ShapeDtypeStruct((B, nq, k), jnp.int32),
    )(qc, rt)


def _gather_chunk(bpw, D):
    for c in range(128, 0, -8):
        if bpw % c == 0 and c * D * 4 <= 200_000 and (bpw // c) % 2 != 1:
            return c
    if bpw * D * 4 <= 200_000:
        return bpw
    raise ValueError((bpw, D))


def _gather_rows(table, idx, force_untiled=False):
    T, D = table.shape
    (M,) = idx.shape
    tiled = M % 4096 == 0 and not force_untiled
    if tiled and D % 128 != 0:
        Dp = (D + 127) // 128 * 128
        table = jnp.pad(table, ((0, 0), (0, Dp - D)))
        D = Dp
    bpw = M // _SC_NW
    chunk = _gather_chunk(bpw, D)
    nchunks = bpw // chunk
    mesh = plsc.VectorSubcoreMesh(core_axis_name="c", subcore_axis_name="s")

    @functools.partial(
        pl.kernel, mesh=mesh,
        out_type=jax.ShapeDtypeStruct((M, D), jnp.float32),
        scratch_types=[pltpu.VMEM((bpw,), jnp.int32),
                       pltpu.VMEM((chunk, D), jnp.float32),
                       pltpu.VMEM((chunk, D), jnp.float32),
                       pltpu.SemaphoreType.DMA,
                       pltpu.SemaphoreType.DMA,
                       pltpu.SemaphoreType.DMA,
                       pltpu.SemaphoreType.DMA],
        compiler_params=pltpu.CompilerParams(use_tc_tiling_on_sc=tiled),
    )
    def kfn(table_hbm, idx_hbm, out_hbm, idx_v, buf0, buf1,
            gsem0, gsem1, ssem0, ssem1):
        wid = lax.axis_index("s") * _SC_NC + lax.axis_index("c")
        base = wid * bpw
        pltpu.sync_copy(idx_hbm.at[pl.ds(base, bpw)], idx_v)

        if nchunks == 1:
            pltpu.async_copy(table_hbm.at[idx_v], buf0, gsem0).wait()
            pltpu.sync_copy(buf0, out_hbm.at[pl.ds(base, bpw)])
            return

        def body(i, carry):
            c0 = 2 * i * chunk
            c1 = c0 + chunk
            g0 = pltpu.async_copy(
                table_hbm.at[idx_v.at[pl.ds(c0, chunk)]], buf0, gsem0)
            g1 = pltpu.async_copy(
                table_hbm.at[idx_v.at[pl.ds(c1, chunk)]], buf1, gsem1)
            g0.wait()
            s0 = pltpu.async_copy(buf0, out_hbm.at[pl.ds(base + c0, chunk)],
                                  ssem0)
            g1.wait()
            s1 = pltpu.async_copy(buf1, out_hbm.at[pl.ds(base + c1, chunk)],
                                  ssem1)
            s0.wait()
            s1.wait()
            return carry

        lax.fori_loop(0, nchunks // 2, body, 0)

    return kfn(table, idx)


def _rowcall(body, rows, row_mults, auxs, out_mult, out_dim, r_total, br,
             out_dtype=jnp.float32):
    grid = (r_total // br,)
    in_specs = []
    for a, m in zip(rows, row_mults):
        in_specs.append(pl.BlockSpec((br * m, a.shape[1]), lambda i: (i, 0)))
    for a in auxs:
        in_specs.append(pl.BlockSpec(a.shape, lambda i: (0,) * a.ndim))
    return _pcall(
        body, grid=grid, in_specs=in_specs,
        out_specs=pl.BlockSpec((br * out_mult, out_dim), lambda i: (i, 0)),
        out_shape=jax.ShapeDtypeStruct((r_total * out_mult, out_dim),
                                       out_dtype),
    )(*rows, *auxs)


def _dot(a, b):
    return jnp.dot(a, b, preferred_element_type=jnp.float32)


def _qkv_body(x_ref, win, bin_, wq, bq_, wkv, bkv, q_ref, kv_ref):
    x = x_ref[...]
    h = _dot(x, win[...]) + bin_[...]
    q_ref[...] = _dot(h, wq[...]) + bq_[...]
    kv_ref[...] = _dot(h, wkv[...]) + bkv[...]


def _attn_tail(dm, knb, vnb, pnb, q_ref, cq_ref, f_ref,
               p1, p1b, p2, p2b, a1, a1b, a2, a2b, lo, lob, o_ref):
    K = K_ATTN
    br = q_ref.shape[0]

    def rep(x):
        return jnp.broadcast_to(
            x[:, None, :], (br, K, x.shape[1])).reshape(br * K, x.shape[1])

    rel = rep(cq_ref[...]) - pnb
    pos = jnp.maximum(_dot(rel, p1[...]) + p1b[...], 0.0)
    pos = _dot(pos, p2[...]) + p2b[...]
    a = rep(q_ref[...]) - knb + pos
    a = jnp.maximum(_dot(a, a1[...]) + a1b[...], 0.0)
    a = _dot(a, a2[...]) + a2b[...]
    a3 = a.reshape(br, K, dm)
    amax = jnp.max(a3, axis=1, keepdims=True)
    e = jnp.exp(a3 - amax)
    w = e / jnp.sum(e, axis=1, keepdims=True)
    v3 = (vnb + pos).reshape(br, K, dm)
    out = jnp.sum(w * v3, axis=1)
    o_ref[...] = f_ref[...] + _dot(out, lo[...]) + lob[...]


def _attn_body(dm, g_ref, q_ref, cq_ref, f_ref,
               p1, p1b, p2, p2b, a1, a1b, a2, a2b, lo, lob, o_ref):
    g = g_ref[...]
    _attn_tail(dm, g[:, :dm], g[:, dm:2 * dm], g[:, 2 * dm:2 * dm + 3],
               q_ref, cq_ref, f_ref,
               p1, p1b, p2, p2b, a1, a1b, a2, a2b, lo, lob, o_ref)


def _attn_body_split(dm, g_ref, pc_ref, q_ref, cq_ref, f_ref,
                     p1, p1b, p2, p2b, a1, a1b, a2, a2b, lo, lob, o_ref):
    g = g_ref[...]
    _attn_tail(dm, g[:, :dm], g[:, dm:2 * dm], pc_ref[...][:, :3],
               q_ref, cq_ref, f_ref,
               p1, p1b, p2, p2b, a1, a1b, a2, a2b, lo, lob, o_ref)


def _attn_body_dense(dm, n, q_ref, cq_ref, f_ref, kv_ref, cb_ref,
                     p1, p1b, p2, p2b, a1, a1b, a2, a2b, lo, lob, o_ref):
    K = K_ATTN
    br = q_ref.shape[0]
    B = br // n
    kv = kv_ref[...].reshape(B, 1, n, 2 * dm)
    kv = jnp.broadcast_to(kv, (B, n, n, 2 * dm)).reshape(br * K, 2 * dm)
    cnb = cb_ref[...].reshape(B, 1, n, 3)
    cnb = jnp.broadcast_to(cnb, (B, n, n, 3)).reshape(br * K, 3)
    _attn_tail(dm, kv[:, :dm], kv[:, dm:2 * dm], cnb,
               q_ref, cq_ref, f_ref,
               p1, p1b, p2, p2b, a1, a1b, a2, a2b, lo, lob, o_ref)


def _down_body(g_ref, w, b, o_ref):
    din = w.shape[0]
    h = jnp.maximum(_dot(g_ref[...][:, :din], w[...]) + b[...], 0.0)
    br, dout = o_ref.shape
    o_ref[...] = jnp.max(h.reshape(br, K_DOWN, dout), axis=1)


def _up_body(dlow, g_ref, cq_ref, fh_ref, ws, bs, o_ref):
    K = K_UP_PAD
    br = cq_ref.shape[0]
    g = g_ref[...]
    nb = g[:, :dlow]
    pnb = g[:, dlow:dlow + 3]
    cqn = jnp.broadcast_to(
        cq_ref[...][:, None, :], (br, K, 3)).reshape(br * K, 3)
    diff = cqn - pnb
    dist = jnp.sum(diff * diff, axis=1, keepdims=True)
    w = 1.0 / (dist + 1e-8)
    w3 = w.reshape(br, K, 1)
    kio = lax.broadcasted_iota(jnp.int32, (br, K, 1), 1)
    w3 = jnp.where(kio < K_UP, w3, 0.0)
    w3 = w3 / jnp.sum(w3, axis=1, keepdims=True)
    interp = jnp.sum(w3 * nb.reshape(br, K, dlow), axis=1)
    o_ref[...] = interp + _dot(fh_ref[...], ws[...]) + bs[...]


def _mlp_body(logsm, x_ref, w1, b1, w2, b2, o_ref):
    h = jnp.maximum(_dot(x_ref[...], w1[...]) + b1[...], 0.0)
    y = _dot(h, w2[...]) + b2[...]
    if logsm:
        m = jnp.max(y, axis=1, keepdims=True)
        e = y - m
        y = e - jnp.log(jnp.sum(jnp.exp(e), axis=1, keepdims=True))
    o_ref[...] = y


def _lin_body(x_ref, w, b, o_ref):
    o_ref[...] = _dot(x_ref[...], w[...]) + b[...]


def _b2(b):
    return b.reshape(1, -1)


def _mlp2(x, mp, logsm=False):
    (w1, b1), (w2, b2) = mp
    r = x.shape[0]
    br = min(r, 1024)
    return _rowcall(functools.partial(_mlp_body, logsm), [x], [1],
                    [w1, _b2(b1), w2, _b2(b2)], 1, w2.shape[1], r, br)


def _lin(x, wb):
    w, b = wb
    r = x.shape[0]
    br = min(r, 1024)
    return _rowcall(_lin_body, [x], [1], [w, _b2(b)], 1, w.shape[1], r, br)


def _pt_block(cb, f, p, idx):
    B, N, _ = cb.shape
    d = f.shape[1]
    R = B * N
    (win, bin_), (wq, bq_) = p['lin_in'], p['q']
    (wk, bk), (wv, bv) = p['k'], p['v']
    wkv = jnp.concatenate([wk, wv], axis=1)
    bkv = jnp.concatenate([bk, bv])
    br = min(R, 512)
    q, kv = _pcall(
        _qkv_body, grid=(R // br,),
        in_specs=[pl.BlockSpec((br, f.shape[1]), lambda i: (i, 0))]
        + [pl.BlockSpec(a.shape, lambda i: (0, 0))
           for a in (win, _b2(bin_), wq, _b2(bq_), wkv, _b2(bkv))],
        out_specs=[pl.BlockSpec((br, d), lambda i: (i, 0)),
                   pl.BlockSpec((br, 2 * d), lambda i: (i, 0))],
        out_shape=[jax.ShapeDtypeStruct((R, d), jnp.float32),
                   jax.ShapeDtypeStruct((R, 2 * d), jnp.float32)],
    )(f, win, _b2(bin_), wq, _b2(bq_), wkv, _b2(bkv))
    cflat = cb.reshape(R, 3)
    (p1, p1b), (p2, p2b) = p['pos1'], p['pos2']
    (a1, a1b), (a2, a2b) = p['attn1'], p['attn2']
    (lo, lob) = p['lin_out']
    wts = [p1, _b2(p1b), p2, _b2(p2b), a1, _b2(a1b), a2, _b2(a2b),
           lo, _b2(lob)]
    br = min(R, 512)
    if N == K_ATTN:
        return _rowcall(
            functools.partial(_attn_body_dense, d, N),
            [q, cflat, f], [1, 1, 1], [kv, cflat] + wts, 1, d, R, br)
    if d == 32:
        table = jnp.concatenate(
            [kv, jnp.pad(cflat, ((0, 0), (0, 13)))], axis=1)
        g = _gather_rows(table, idx.reshape(-1))
        return _rowcall(
            functools.partial(_attn_body, d), [g, q, cflat, f],
            [K_ATTN, 1, 1, 1], wts, 1, d, R, br)
    g = _gather_rows(kv, idx.reshape(-1))
    pc = _gather_rows(jnp.pad(cflat, ((0, 0), (0, 13))), idx.reshape(-1),
                      force_untiled=True)
    return _rowcall(
        functools.partial(_attn_body_split, d), [g, pc, q, cflat, f],
        [K_ATTN, K_ATTN, 1, 1, 1], wts, 1, d, R, br)


def _tdown(cb, f, wb, n_out, idx):
    B = cb.shape[0]
    nc = cb[:, :n_out]
    g = _gather_rows(f, idx.reshape(-1))
    w, b = wb
    r = B * n_out
    br = min(r, 512)
    out = _rowcall(_down_body, [g], [K_DOWN], [w, _b2(b)],
                   1, w.shape[1], r, br)
    return nc, out


def _tup(cl, f_low, ch, f_high, p, idx3):
    B, nl, _ = cl.shape
    nh = ch.shape[1]
    fl = _lin(f_low, p['low'])
    d = fl.shape[1]
    idx8 = jnp.concatenate(
        [idx3, jnp.broadcast_to(idx3[..., :1], (B, nh, K_UP_PAD - K_UP))],
        axis=2)
    table = jnp.concatenate(
        [fl, jnp.pad(cl.reshape(B * nl, 3), ((0, 0), (0, 13)))], axis=1)
    g = _gather_rows(table, idx8.reshape(-1))
    ws, bs = p['skip']
    r = B * nh
    br = min(r, 512)
    return _rowcall(functools.partial(_up_body, d),
                    [g, ch.reshape(r, 3), f_high], [K_UP_PAD, 1, 1],
                    [ws, _b2(bs)], 1, d, r, br)


def kernel(coords, features, params):
    B, N, _ = coords.shape
    c0 = coords
    c1, c2, c3, c4 = (c0[:, :N // 4], c0[:, :N // 16],
                      c0[:, :N // 64], c0[:, :N // 256])
    n4 = N // 256
    base = jnp.arange(n4, dtype=jnp.int32)[None, None, :]
    off = jnp.arange(B, dtype=jnp.int32)[:, None, None] * n4
    idx_a4 = jnp.broadcast_to(base + off, (B, n4, n4))
    idx_a0 = _knn(c0, c0, K_ATTN)
    idx_a1 = _knn(c1, c1, K_ATTN)
    idx_a2 = _knn(c2, c2, K_ATTN)
    idx_a3 = _knn(c3, c3, K_ATTN)
    idx_d1 = idx_a0[:, :N // 4]
    idx_d2 = idx_a1[:, :N // 16]
    idx_d3 = idx_a2[:, :N // 64]
    idx_d4 = idx_a3[:, :N // 256]
    idx_u6 = _knn(c3, c4, K_UP)
    idx_u7 = _knn(c2, c3, K_UP)
    idx_u8 = _knn(c1, c2, K_UP)
    idx_u9 = _knn(c0, c1, K_UP)

    f = features.reshape(B * N, 3)
    f0 = _mlp2(f, params['mlp0'])
    f0 = _pt_block(c0, f0, params['pt0'], idx_a0)
    _, f1 = _tdown(c0, f0, params['td1'], N // 4, idx_d1)
    f1 = _pt_block(c1, f1, params['pt1'], idx_a1)
    _, f2 = _tdown(c1, f1, params['td2'], N // 16, idx_d2)
    f2 = _pt_block(c2, f2, params['pt2'], idx_a2)
    _, f3 = _tdown(c2, f2, params['td3'], N // 64, idx_d3)
    f3 = _pt_block(c3, f3, params['pt3'], idx_a3)
    _, f4 = _tdown(c3, f3, params['td4'], N // 256, idx_d4)
    f4 = _pt_block(c4, f4, params['pt4'], idx_a4)
    f5 = _mlp2(f4, params['mlp2'])
    f6 = _pt_block(c4, f5, params['pt5'], idx_a4)
    f7 = _tup(c4, f6, c3, f3, params['tu6'], idx_u6)
    f7 = _pt_block(c3, f7, params['pt6'], idx_a3)
    f8 = _tup(c3, f7, c2, f2, params['tu7'], idx_u7)
    f8 = _pt_block(c2, f8, params['pt7'], idx_a2)
    f9 = _tup(c2, f8, c1, f1, params['tu8'], idx_u8)
    f9 = _pt_block(c1, f9, params['pt8'], idx_a1)
    f10 = _tup(c1, f9, c0, f0, params['tu9'], idx_u9)
    f10 = _pt_block(c0, f10, params['pt9'], idx_a0)
    logits = _mlp2(f10, params['mlp3'], logsm=True)
    return logits.reshape(B, N, -1)


# speedup vs baseline: 1.2480x; 1.0028x over previous
import functools

import jax
import jax.numpy as jnp
from jax import lax
from jax.experimental import pallas as pl
from jax.experimental.pallas import tpu as pltpu
from jax.experimental.pallas import tpu_sc as plsc

_pcall = pl.pallas_call

K_ATTN = 16
K_DOWN = 16
K_UP = 3
K_UP_PAD = 8

_SC_NC = 2
_SC_NS = 16
_SC_NW = _SC_NC * _SC_NS


def _knn_body(k, nr, q_ref, rt_ref, o_ref):
    b = pl.program_id(0)
    q = q_ref[0]
    rt = rt_ref[0]
    qq = jnp.sum(q * q, axis=1, keepdims=True)
    rr = jnp.sum(rt * rt, axis=0, keepdims=True)
    cross = jnp.dot(q, rt)
    d = (qq - 2.0 * cross) + rr
    iota = lax.broadcasted_iota(jnp.int32, d.shape, 1)
    cols = []
    for _ in range(k):
        am = jnp.argmin(d, axis=1).astype(jnp.int32)[:, None]
        cols.append(am)
        d = jnp.where(iota == am, jnp.float32(jnp.inf), d)
    idx = jnp.concatenate(cols, axis=1)
    o_ref[0] = idx + b * nr


def _knn(qc, rc, k):
    B, nq, _ = qc.shape
    nr = rc.shape[1]
    rt = jnp.swapaxes(rc, 1, 2)
    bq = min(nq, 256)
    return _pcall(
        functools.partial(_knn_body, k, nr),
        grid=(B, nq // bq),
        in_specs=[pl.BlockSpec((1, bq, 3), lambda b, i: (b, i, 0)),
                  pl.BlockSpec((1, 3, nr), lambda b, i: (b, 0, 0))],
        out_specs=pl.BlockSpec((1, bq, k), lambda b, i: (b, i, 0)),
        out_shape=jax.ShapeDtypeStruct((B, nq, k), jnp.int32),
    )(qc, rt)


def _gather_chunk(bpw, D):
    for c in range(128, 0, -8):
        if bpw % c == 0 and c * D * 4 <= 200_000 and (bpw // c) % 2 != 1:
            return c
    if bpw * D * 4 <= 200_000:
        return bpw
    raise ValueError((bpw, D))


def _gather_rows(table, idx, force_untiled=False):
    T, D = table.shape
    (M,) = idx.shape
    tiled = M % 4096 == 0 and not force_untiled
    if tiled and D % 128 != 0:
        Dp = (D + 127) // 128 * 128
        table = jnp.pad(table, ((0, 0), (0, Dp - D)))
        D = Dp
    bpw = M // _SC_NW
    chunk = _gather_chunk(bpw, D)
    nchunks = bpw // chunk
    mesh = plsc.VectorSubcoreMesh(core_axis_name="c", subcore_axis_name="s")

    @functools.partial(
        pl.kernel, mesh=mesh,
        out_type=jax.ShapeDtypeStruct((M, D), jnp.float32),
        scratch_types=[pltpu.VMEM((bpw,), jnp.int32),
                       pltpu.VMEM((chunk, D), jnp.float32),
                       pltpu.VMEM((chunk, D), jnp.float32),
                       pltpu.SemaphoreType.DMA,
                       pltpu.SemaphoreType.DMA,
                       pltpu.SemaphoreType.DMA,
                       pltpu.SemaphoreType.DMA],
        compiler_params=pltpu.CompilerParams(use_tc_tiling_on_sc=tiled),
        cost_estimate=pl.CostEstimate(
            flops=0, transcendentals=0,
            bytes_accessed=2 * M * D * 4 + M * 4),
    )
    def kfn(table_hbm, idx_hbm, out_hbm, idx_v, buf0, buf1,
            gsem0, gsem1, ssem0, ssem1):
        wid = lax.axis_index("s") * _SC_NC + lax.axis_index("c")
        base = wid * bpw
        pltpu.sync_copy(idx_hbm.at[pl.ds(base, bpw)], idx_v)

        if nchunks == 1:
            pltpu.async_copy(table_hbm.at[idx_v], buf0, gsem0).wait()
            pltpu.sync_copy(buf0, out_hbm.at[pl.ds(base, bpw)])
            return

        def body(i, carry):
            c0 = 2 * i * chunk
            c1 = c0 + chunk
            g0 = pltpu.async_copy(
                table_hbm.at[idx_v.at[pl.ds(c0, chunk)]], buf0, gsem0)
            g1 = pltpu.async_copy(
                table_hbm.at[idx_v.at[pl.ds(c1, chunk)]], buf1, gsem1)
            g0.wait()
            s0 = pltpu.async_copy(buf0, out_hbm.at[pl.ds(base + c0, chunk)],
                                  ssem0)
            g1.wait()
            s1 = pltpu.async_copy(buf1, out_hbm.at[pl.ds(base + c1, chunk)],
                                  ssem1)
            s0.wait()
            s1.wait()
            return carry

        lax.fori_loop(0, nchunks // 2, body, 0)

    return kfn(table, idx)


def _rowcall(body, rows, row_mults, auxs, out_mult, out_dim, r_total, br,
             out_dtype=jnp.float32):
    grid = (r_total // br,)
    in_specs = []
    for a, m in zip(rows, row_mults):
        in_specs.append(pl.BlockSpec((br * m, a.shape[1]), lambda i: (i, 0)))
    for a in auxs:
        in_specs.append(pl.BlockSpec(a.shape, lambda i: (0,) * a.ndim))
    return _pcall(
        body, grid=grid, in_specs=in_specs,
        out_specs=pl.BlockSpec((br * out_mult, out_dim), lambda i: (i, 0)),
        out_shape=jax.ShapeDtypeStruct((r_total * out_mult, out_dim),
                                       out_dtype),
    )(*rows, *auxs)


def _dot(a, b):
    return jnp.dot(a, b, preferred_element_type=jnp.float32)


def _qkv_body(x_ref, win, bin_, wq, bq_, wkv, bkv, q_ref, kv_ref):
    x = x_ref[...]
    h = _dot(x, win[...]) + bin_[...]
    q_ref[...] = _dot(h, wq[...]) + bq_[...]
    kv_ref[...] = _dot(h, wkv[...]) + bkv[...]


def _attn_tail(dm, knb, vnb, pnb, q_ref, cq_ref, f_ref,
               p1, p1b, p2, p2b, a1, a1b, a2, a2b, lo, lob, o_ref):
    K = K_ATTN
    br = q_ref.shape[0]

    def rep(x):
        return jnp.broadcast_to(
            x[:, None, :], (br, K, x.shape[1])).reshape(br * K, x.shape[1])

    rel = rep(cq_ref[...]) - pnb
    pos = jnp.maximum(_dot(rel, p1[...]) + p1b[...], 0.0)
    pos = _dot(pos, p2[...]) + p2b[...]
    a = rep(q_ref[...]) - knb + pos
    a = jnp.maximum(_dot(a, a1[...]) + a1b[...], 0.0)
    a = _dot(a, a2[...]) + a2b[...]
    a3 = a.reshape(br, K, dm)
    amax = jnp.max(a3, axis=1, keepdims=True)
    e = jnp.exp(a3 - amax)
    w = e / jnp.sum(e, axis=1, keepdims=True)
    v3 = (vnb + pos).reshape(br, K, dm)
    out = jnp.sum(w * v3, axis=1)
    o_ref[...] = f_ref[...] + _dot(out, lo[...]) + lob[...]


def _attn_body(dm, g_ref, q_ref, cq_ref, f_ref,
               p1, p1b, p2, p2b, a1, a1b, a2, a2b, lo, lob, o_ref):
    g = g_ref[...]
    _attn_tail(dm, g[:, :dm], g[:, dm:2 * dm], g[:, 2 * dm:2 * dm + 3],
               q_ref, cq_ref, f_ref,
               p1, p1b, p2, p2b, a1, a1b, a2, a2b, lo, lob, o_ref)


def _attn_body_split(dm, g_ref, pc_ref, q_ref, cq_ref, f_ref,
                     p1, p1b, p2, p2b, a1, a1b, a2, a2b, lo, lob, o_ref):
    g = g_ref[...]
    _attn_tail(dm, g[:, :dm], g[:, dm:2 * dm], pc_ref[...][:, :3],
               q_ref, cq_ref, f_ref,
               p1, p1b, p2, p2b, a1, a1b, a2, a2b, lo, lob, o_ref)


def _attn_body_dense(dm, n, q_ref, cq_ref, f_ref, kv_ref, cb_ref,
                     p1, p1b, p2, p2b, a1, a1b, a2, a2b, lo, lob, o_ref):
    K = K_ATTN
    br = q_ref.shape[0]
    B = br // n
    kv = kv_ref[...].reshape(B, 1, n, 2 * dm)
    kv = jnp.broadcast_to(kv, (B, n, n, 2 * dm)).reshape(br * K, 2 * dm)
    cnb = cb_ref[...].reshape(B, 1, n, 3)
    cnb = jnp.broadcast_to(cnb, (B, n, n, 3)).reshape(br * K, 3)
    _attn_tail(dm, kv[:, :dm], kv[:, dm:2 * dm], cnb,
               q_ref, cq_ref, f_ref,
               p1, p1b, p2, p2b, a1, a1b, a2, a2b, lo, lob, o_ref)


def _down_body(g_ref, w, b, o_ref):
    din = w.shape[0]
    h = jnp.maximum(_dot(g_ref[...][:, :din], w[...]) + b[...], 0.0)
    br, dout = o_ref.shape
    o_ref[...] = jnp.max(h.reshape(br, K_DOWN, dout), axis=1)


def _up_body(dlow, g_ref, cq_ref, fh_ref, ws, bs, o_ref):
    K = K_UP_PAD
    br = cq_ref.shape[0]
    g = g_ref[...]
    nb = g[:, :dlow]
    pnb = g[:, dlow:dlow + 3]
    cqn = jnp.broadcast_to(
        cq_ref[...][:, None, :], (br, K, 3)).reshape(br * K, 3)
    diff = cqn - pnb
    dist = jnp.sum(diff * diff, axis=1, keepdims=True)
    w = 1.0 / (dist + 1e-8)
    w3 = w.reshape(br, K, 1)
    kio = lax.broadcasted_iota(jnp.int32, (br, K, 1), 1)
    w3 = jnp.where(kio < K_UP, w3, 0.0)
    w3 = w3 / jnp.sum(w3, axis=1, keepdims=True)
    interp = jnp.sum(w3 * nb.reshape(br, K, dlow), axis=1)
    o_ref[...] = interp + _dot(fh_ref[...], ws[...]) + bs[...]


def _mlp_body(logsm, x_ref, w1, b1, w2, b2, o_ref):
    h = jnp.maximum(_dot(x_ref[...], w1[...]) + b1[...], 0.0)
    y = _dot(h, w2[...]) + b2[...]
    if logsm:
        m = jnp.max(y, axis=1, keepdims=True)
        e = y - m
        y = e - jnp.log(jnp.sum(jnp.exp(e), axis=1, keepdims=True))
    o_ref[...] = y


def _lin_body(x_ref, w, b, o_ref):
    o_ref[...] = _dot(x_ref[...], w[...]) + b[...]


def _b2(b):
    return b.reshape(1, -1)


def _mlp2(x, mp, logsm=False):
    (w1, b1), (w2, b2) = mp
    r = x.shape[0]
    br = min(r, 1024)
    return _rowcall(functools.partial(_mlp_body, logsm), [x], [1],
                    [w1, _b2(b1), w2, _b2(b2)], 1, w2.shape[1], r, br)


def _lin(x, wb):
    w, b = wb
    r = x.shape[0]
    br = min(r, 1024)
    return _rowcall(_lin_body, [x], [1], [w, _b2(b)], 1, w.shape[1], r, br)


def _pt_block(cb, f, p, idx):
    B, N, _ = cb.shape
    d = f.shape[1]
    R = B * N
    (win, bin_), (wq, bq_) = p['lin_in'], p['q']
    (wk, bk), (wv, bv) = p['k'], p['v']
    wkv = jnp.concatenate([wk, wv], axis=1)
    bkv = jnp.concatenate([bk, bv])
    br = min(R, 512)
    q, kv = _pcall(
        _qkv_body, grid=(R // br,),
        in_specs=[pl.BlockSpec((br, f.shape[1]), lambda i: (i, 0))]
        + [pl.BlockSpec(a.shape, lambda i: (0, 0))
           for a in (win, _b2(bin_), wq, _b2(bq_), wkv, _b2(bkv))],
        out_specs=[pl.BlockSpec((br, d), lambda i: (i, 0)),
                   pl.BlockSpec((br, 2 * d), lambda i: (i, 0))],
        out_shape=[jax.ShapeDtypeStruct((R, d), jnp.float32),
                   jax.ShapeDtypeStruct((R, 2 * d), jnp.float32)],
    )(f, win, _b2(bin_), wq, _b2(bq_), wkv, _b2(bkv))
    cflat = cb.reshape(R, 3)
    (p1, p1b), (p2, p2b) = p['pos1'], p['pos2']
    (a1, a1b), (a2, a2b) = p['attn1'], p['attn2']
    (lo, lob) = p['lin_out']
    wts = [p1, _b2(p1b), p2, _b2(p2b), a1, _b2(a1b), a2, _b2(a2b),
           lo, _b2(lob)]
    br = min(R, 512)
    if N == K_ATTN:
        return _rowcall(
            functools.partial(_attn_body_dense, d, N),
            [q, cflat, f], [1, 1, 1], [kv, cflat] + wts, 1, d, R, br)
    if d == 32:
        table = jnp.concatenate(
            [kv, jnp.pad(cflat, ((0, 0), (0, 13)))], axis=1)
        g = _gather_rows(table, idx.reshape(-1))
        return _rowcall(
            functools.partial(_attn_body, d), [g, q, cflat, f],
            [K_ATTN, 1, 1, 1], wts, 1, d, R, br)
    g = _gather_rows(kv, idx.reshape(-1))
    pc = _gather_rows(jnp.pad(cflat, ((0, 0), (0, 13))), idx.reshape(-1),
                      force_untiled=True)
    return _rowcall(
        functools.partial(_attn_body_split, d), [g, pc, q, cflat, f],
        [K_ATTN, K_ATTN, 1, 1, 1], wts, 1, d, R, br)


def _tdown(cb, f, wb, n_out, idx):
    B = cb.shape[0]
    nc = cb[:, :n_out]
    g = _gather_rows(f, idx.reshape(-1))
    w, b = wb
    r = B * n_out
    br = min(r, 512)
    out = _rowcall(_down_body, [g], [K_DOWN], [w, _b2(b)],
                   1, w.shape[1], r, br)
    return nc, out


def _tup(cl, f_low, ch, f_high, p, idx3):
    B, nl, _ = cl.shape
    nh = ch.shape[1]
    fl = _lin(f_low, p['low'])
    d = fl.shape[1]
    idx8 = jnp.concatenate(
        [idx3, jnp.broadcast_to(idx3[..., :1], (B, nh, K_UP_PAD - K_UP))],
        axis=2)
    table = jnp.concatenate(
        [fl, jnp.pad(cl.reshape(B * nl, 3), ((0, 0), (0, 13)))], axis=1)
    g = _gather_rows(table, idx8.reshape(-1))
    ws, bs = p['skip']
    r = B * nh
    br = min(r, 512)
    return _rowcall(functools.partial(_up_body, d),
                    [g, ch.reshape(r, 3), f_high], [K_UP_PAD, 1, 1],
                    [ws, _b2(bs)], 1, d, r, br)


def kernel(coords, features, params):
    B, N, _ = coords.shape
    c0 = coords
    c1, c2, c3, c4 = (c0[:, :N // 4], c0[:, :N // 16],
                      c0[:, :N // 64], c0[:, :N // 256])
    n4 = N // 256
    base = jnp.arange(n4, dtype=jnp.int32)[None, None, :]
    off = jnp.arange(B, dtype=jnp.int32)[:, None, None] * n4
    idx_a4 = jnp.broadcast_to(base + off, (B, n4, n4))
    idx_a0 = _knn(c0, c0, K_ATTN)
    idx_a1 = _knn(c1, c1, K_ATTN)
    idx_a2 = _knn(c2, c2, K_ATTN)
    idx_a3 = _knn(c3, c3, K_ATTN)
    idx_d1 = idx_a0[:, :N // 4]
    idx_d2 = idx_a1[:, :N // 16]
    idx_d3 = idx_a2[:, :N // 64]
    idx_d4 = idx_a3[:, :N // 256]
    idx_u6 = _knn(c3, c4, K_UP)
    idx_u7 = _knn(c2, c3, K_UP)
    idx_u8 = _knn(c1, c2, K_UP)
    idx_u9 = _knn(c0, c1, K_UP)

    f = features.reshape(B * N, 3)
    f0 = _mlp2(f, params['mlp0'])
    f0 = _pt_block(c0, f0, params['pt0'], idx_a0)
    _, f1 = _tdown(c0, f0, params['td1'], N // 4, idx_d1)
    f1 = _pt_block(c1, f1, params['pt1'], idx_a1)
    _, f2 = _tdown(c1, f1, params['td2'], N // 16, idx_d2)
    f2 = _pt_block(c2, f2, params['pt2'], idx_a2)
    _, f3 = _tdown(c2, f2, params['td3'], N // 64, idx_d3)
    f3 = _pt_block(c3, f3, params['pt3'], idx_a3)
    _, f4 = _tdown(c3, f3, params['td4'], N // 256, idx_d4)
    f4 = _pt_block(c4, f4, params['pt4'], idx_a4)
    f5 = _mlp2(f4, params['mlp2'])
    f6 = _pt_block(c4, f5, params['pt5'], idx_a4)
    f7 = _tup(c4, f6, c3, f3, params['tu6'], idx_u6)
    f7 = _pt_block(c3, f7, params['pt6'], idx_a3)
    f8 = _tup(c3, f7, c2, f2, params['tu7'], idx_u7)
    f8 = _pt_block(c2, f8, params['pt7'], idx_a2)
    f9 = _tup(c2, f8, c1, f1, params['tu8'], idx_u8)
    f9 = _pt_block(c1, f9, params['pt8'], idx_a1)
    f10 = _tup(c1, f9, c0, f0, params['tu9'], idx_u9)
    f10 = _pt_block(c0, f10, params['pt9'], idx_a0)
    logits = _mlp2(f10, params['mlp3'], logsm=True)
    return logits.reshape(B, N, -1)


# speedup vs baseline: 1.2650x; 1.0136x over previous
import functools

import jax
import jax.numpy as jnp
from jax import lax
from jax.experimental import pallas as pl
from jax.experimental.pallas import tpu as pltpu
from jax.experimental.pallas import tpu_sc as plsc

_pcall = pl.pallas_call

K_ATTN = 16
K_DOWN = 16
K_UP = 3
K_UP_PAD = 8

_SC_NC = 2
_SC_NS = 16
_SC_NW = _SC_NC * _SC_NS


def _knn_body(k, nr, q_ref, rt_ref, o_ref):
    b = pl.program_id(0)
    q = q_ref[0]
    rt = rt_ref[0]
    qq = jnp.sum(q * q, axis=1, keepdims=True)
    rr = jnp.sum(rt * rt, axis=0, keepdims=True)
    cross = jnp.dot(q, rt)
    d = (qq - 2.0 * cross) + rr
    iota = lax.broadcasted_iota(jnp.int32, d.shape, 1)
    cols = []
    for _ in range(k):
        am = jnp.argmin(d, axis=1).astype(jnp.int32)[:, None]
        cols.append(am)
        d = jnp.where(iota == am, jnp.float32(jnp.inf), d)
    idx = jnp.concatenate(cols, axis=1)
    o_ref[0] = idx + b * nr


def _knn(qc, rc, k):
    B, nq, _ = qc.shape
    nr = rc.shape[1]
    rt = jnp.swapaxes(rc, 1, 2)
    bq = min(nq, 512)
    return _pcall(
        functools.partial(_knn_body, k, nr),
        grid=(B, nq // bq),
        in_specs=[pl.BlockSpec((1, bq, 3), lambda b, i: (b, i, 0)),
                  pl.BlockSpec((1, 3, nr), lambda b, i: (b, 0, 0))],
        out_specs=pl.BlockSpec((1, bq, k), lambda b, i: (b, i, 0)),
        out_shape=jax.ShapeDtypeStruct((B, nq, k), jnp.int32),
    )(qc, rt)


def _gather_chunk(bpw, D):
    for c in range(128, 0, -8):
        if bpw % c == 0 and c * D * 4 <= 200_000 and (bpw // c) % 2 != 1:
            return c
    if bpw * D * 4 <= 200_000:
        return bpw
    raise ValueError((bpw, D))


def _gather_rows(table, idx, force_untiled=False):
    T, D = table.shape
    (M,) = idx.shape
    tiled = M % 4096 == 0 and not force_untiled
    if tiled and D % 128 != 0:
        Dp = (D + 127) // 128 * 128
        table = jnp.pad(table, ((0, 0), (0, Dp - D)))
        D = Dp
    bpw = M // _SC_NW
    chunk = _gather_chunk(bpw, D)
    nchunks = bpw // chunk
    mesh = plsc.VectorSubcoreMesh(core_axis_name="c", subcore_axis_name="s")

    @functools.partial(
        pl.kernel, mesh=mesh,
        out_type=jax.ShapeDtypeStruct((M, D), jnp.float32),
        scratch_types=[pltpu.VMEM((bpw,), jnp.int32),
                       pltpu.VMEM((chunk, D), jnp.float32),
                       pltpu.VMEM((chunk, D), jnp.float32),
                       pltpu.SemaphoreType.DMA,
                       pltpu.SemaphoreType.DMA,
                       pltpu.SemaphoreType.DMA,
                       pltpu.SemaphoreType.DMA],
        compiler_params=pltpu.CompilerParams(use_tc_tiling_on_sc=tiled),
        cost_estimate=pl.CostEstimate(
            flops=0, transcendentals=0,
            bytes_accessed=2 * M * D * 4 + M * 4),
    )
    def kfn(table_hbm, idx_hbm, out_hbm, idx_v, buf0, buf1,
            gsem0, gsem1, ssem0, ssem1):
        wid = lax.axis_index("s") * _SC_NC + lax.axis_index("c")
        base = wid * bpw
        pltpu.sync_copy(idx_hbm.at[pl.ds(base, bpw)], idx_v)

        if nchunks == 1:
            pltpu.async_copy(table_hbm.at[idx_v], buf0, gsem0).wait()
            pltpu.sync_copy(buf0, out_hbm.at[pl.ds(base, bpw)])
            return

        def body(i, carry):
            c0 = 2 * i * chunk
            c1 = c0 + chunk
            g0 = pltpu.async_copy(
                table_hbm.at[idx_v.at[pl.ds(c0, chunk)]], buf0, gsem0)
            g1 = pltpu.async_copy(
                table_hbm.at[idx_v.at[pl.ds(c1, chunk)]], buf1, gsem1)
            g0.wait()
            s0 = pltpu.async_copy(buf0, out_hbm.at[pl.ds(base + c0, chunk)],
                                  ssem0)
            g1.wait()
            s1 = pltpu.async_copy(buf1, out_hbm.at[pl.ds(base + c1, chunk)],
                                  ssem1)
            s0.wait()
            s1.wait()
            return carry

        lax.fori_loop(0, nchunks // 2, body, 0)

    return kfn(table, idx)


def _rowcall(body, rows, row_mults, auxs, out_mult, out_dim, r_total, br,
             out_dtype=jnp.float32):
    grid = (r_total // br,)
    in_specs = []
    for a, m in zip(rows, row_mults):
        in_specs.append(pl.BlockSpec((br * m, a.shape[1]), lambda i: (i, 0)))
    for a in auxs:
        in_specs.append(pl.BlockSpec(a.shape, lambda i: (0,) * a.ndim))
    return _pcall(
        body, grid=grid, in_specs=in_specs,
        out_specs=pl.BlockSpec((br * out_mult, out_dim), lambda i: (i, 0)),
        out_shape=jax.ShapeDtypeStruct((r_total * out_mult, out_dim),
                                       out_dtype),
    )(*rows, *auxs)


def _dot(a, b):
    return jnp.dot(a, b, preferred_element_type=jnp.float32)


def _qkv_body(x_ref, win, bin_, wq, bq_, wkv, bkv, q_ref, kv_ref):
    x = x_ref[...]
    h = _dot(x, win[...]) + bin_[...]
    q_ref[...] = _dot(h, wq[...]) + bq_[...]
    kv_ref[...] = _dot(h, wkv[...]) + bkv[...]


def _attn_tail(dm, knb, vnb, pnb, q_ref, cq_ref, f_ref,
               p1, p1b, p2, p2b, a1, a1b, a2, a2b, lo, lob, o_ref):
    K = K_ATTN
    br = q_ref.shape[0]

    def rep(x):
        return jnp.broadcast_to(
            x[:, None, :], (br, K, x.shape[1])).reshape(br * K, x.shape[1])

    rel = rep(cq_ref[...]) - pnb
    pos = jnp.maximum(_dot(rel, p1[...]) + p1b[...], 0.0)
    pos = _dot(pos, p2[...]) + p2b[...]
    a = rep(q_ref[...]) - knb + pos
    a = jnp.maximum(_dot(a, a1[...]) + a1b[...], 0.0)
    a = _dot(a, a2[...]) + a2b[...]
    a3 = a.reshape(br, K, dm)
    amax = jnp.max(a3, axis=1, keepdims=True)
    e = jnp.exp(a3 - amax)
    w = e / jnp.sum(e, axis=1, keepdims=True)
    v3 = (vnb + pos).reshape(br, K, dm)
    out = jnp.sum(w * v3, axis=1)
    o_ref[...] = f_ref[...] + _dot(out, lo[...]) + lob[...]


def _attn_body(dm, g_ref, q_ref, cq_ref, f_ref,
               p1, p1b, p2, p2b, a1, a1b, a2, a2b, lo, lob, o_ref):
    g = g_ref[...]
    _attn_tail(dm, g[:, :dm], g[:, dm:2 * dm], g[:, 2 * dm:2 * dm + 3],
               q_ref, cq_ref, f_ref,
               p1, p1b, p2, p2b, a1, a1b, a2, a2b, lo, lob, o_ref)


def _attn_body_split(dm, g_ref, pc_ref, q_ref, cq_ref, f_ref,
                     p1, p1b, p2, p2b, a1, a1b, a2, a2b, lo, lob, o_ref):
    g = g_ref[...]
    _attn_tail(dm, g[:, :dm], g[:, dm:2 * dm], pc_ref[...][:, :3],
               q_ref, cq_ref, f_ref,
               p1, p1b, p2, p2b, a1, a1b, a2, a2b, lo, lob, o_ref)


def _attn_body_dense(dm, n, q_ref, cq_ref, f_ref, kv_ref, cb_ref,
                     p1, p1b, p2, p2b, a1, a1b, a2, a2b, lo, lob, o_ref):
    K = K_ATTN
    br = q_ref.shape[0]
    B = br // n
    kv = kv_ref[...].reshape(B, 1, n, 2 * dm)
    kv = jnp.broadcast_to(kv, (B, n, n, 2 * dm)).reshape(br * K, 2 * dm)
    cnb = cb_ref[...].reshape(B, 1, n, 3)
    cnb = jnp.broadcast_to(cnb, (B, n, n, 3)).reshape(br * K, 3)
    _attn_tail(dm, kv[:, :dm], kv[:, dm:2 * dm], cnb,
               q_ref, cq_ref, f_ref,
               p1, p1b, p2, p2b, a1, a1b, a2, a2b, lo, lob, o_ref)


def _down_body(g_ref, w, b, o_ref):
    din = w.shape[0]
    h = jnp.maximum(_dot(g_ref[...][:, :din], w[...]) + b[...], 0.0)
    br, dout = o_ref.shape
    o_ref[...] = jnp.max(h.reshape(br, K_DOWN, dout), axis=1)


def _up_body(dlow, g_ref, cq_ref, fh_ref, ws, bs, o_ref):
    K = K_UP_PAD
    br = cq_ref.shape[0]
    g = g_ref[...]
    nb = g[:, :dlow]
    pnb = g[:, dlow:dlow + 3]
    cqn = jnp.broadcast_to(
        cq_ref[...][:, None, :], (br, K, 3)).reshape(br * K, 3)
    diff = cqn - pnb
    dist = jnp.sum(diff * diff, axis=1, keepdims=True)
    w = 1.0 / (dist + 1e-8)
    w3 = w.reshape(br, K, 1)
    kio = lax.broadcasted_iota(jnp.int32, (br, K, 1), 1)
    w3 = jnp.where(kio < K_UP, w3, 0.0)
    w3 = w3 / jnp.sum(w3, axis=1, keepdims=True)
    interp = jnp.sum(w3 * nb.reshape(br, K, dlow), axis=1)
    o_ref[...] = interp + _dot(fh_ref[...], ws[...]) + bs[...]


def _mlp_body(logsm, x_ref, w1, b1, w2, b2, o_ref):
    h = jnp.maximum(_dot(x_ref[...], w1[...]) + b1[...], 0.0)
    y = _dot(h, w2[...]) + b2[...]
    if logsm:
        m = jnp.max(y, axis=1, keepdims=True)
        e = y - m
        y = e - jnp.log(jnp.sum(jnp.exp(e), axis=1, keepdims=True))
    o_ref[...] = y


def _lin_body(x_ref, w, b, o_ref):
    o_ref[...] = _dot(x_ref[...], w[...]) + b[...]


def _b2(b):
    return b.reshape(1, -1)


def _mlp2(x, mp, logsm=False):
    (w1, b1), (w2, b2) = mp
    r = x.shape[0]
    br = min(r, 1024)
    return _rowcall(functools.partial(_mlp_body, logsm), [x], [1],
                    [w1, _b2(b1), w2, _b2(b2)], 1, w2.shape[1], r, br)


def _lin(x, wb):
    w, b = wb
    r = x.shape[0]
    br = min(r, 1024)
    return _rowcall(_lin_body, [x], [1], [w, _b2(b)], 1, w.shape[1], r, br)


def _pt_block(cb, f, p, idx):
    B, N, _ = cb.shape
    d = f.shape[1]
    R = B * N
    (win, bin_), (wq, bq_) = p['lin_in'], p['q']
    (wk, bk), (wv, bv) = p['k'], p['v']
    wkv = jnp.concatenate([wk, wv], axis=1)
    bkv = jnp.concatenate([bk, bv])
    br = min(R, 512)
    q, kv = _pcall(
        _qkv_body, grid=(R // br,),
        in_specs=[pl.BlockSpec((br, f.shape[1]), lambda i: (i, 0))]
        + [pl.BlockSpec(a.shape, lambda i: (0, 0))
           for a in (win, _b2(bin_), wq, _b2(bq_), wkv, _b2(bkv))],
        out_specs=[pl.BlockSpec((br, d), lambda i: (i, 0)),
                   pl.BlockSpec((br, 2 * d), lambda i: (i, 0))],
        out_shape=[jax.ShapeDtypeStruct((R, d), jnp.float32),
                   jax.ShapeDtypeStruct((R, 2 * d), jnp.float32)],
    )(f, win, _b2(bin_), wq, _b2(bq_), wkv, _b2(bkv))
    cflat = cb.reshape(R, 3)
    (p1, p1b), (p2, p2b) = p['pos1'], p['pos2']
    (a1, a1b), (a2, a2b) = p['attn1'], p['attn2']
    (lo, lob) = p['lin_out']
    wts = [p1, _b2(p1b), p2, _b2(p2b), a1, _b2(a1b), a2, _b2(a2b),
           lo, _b2(lob)]
    br = min(R, 512)
    if N == K_ATTN:
        return _rowcall(
            functools.partial(_attn_body_dense, d, N),
            [q, cflat, f], [1, 1, 1], [kv, cflat] + wts, 1, d, R, br)
    if d == 32:
        table = jnp.concatenate(
            [kv, jnp.pad(cflat, ((0, 0), (0, 13)))], axis=1)
        g = _gather_rows(table, idx.reshape(-1))
        return _rowcall(
            functools.partial(_attn_body, d), [g, q, cflat, f],
            [K_ATTN, 1, 1, 1], wts, 1, d, R, br)
    g = _gather_rows(kv, idx.reshape(-1))
    pc = _gather_rows(jnp.pad(cflat, ((0, 0), (0, 13))), idx.reshape(-1),
                      force_untiled=True)
    return _rowcall(
        functools.partial(_attn_body_split, d), [g, pc, q, cflat, f],
        [K_ATTN, K_ATTN, 1, 1, 1], wts, 1, d, R, br)


def _tdown(cb, f, wb, n_out, idx):
    B = cb.shape[0]
    nc = cb[:, :n_out]
    g = _gather_rows(f, idx.reshape(-1))
    w, b = wb
    r = B * n_out
    br = min(r, 512)
    out = _rowcall(_down_body, [g], [K_DOWN], [w, _b2(b)],
                   1, w.shape[1], r, br)
    return nc, out


def _tup(cl, f_low, ch, f_high, p, idx3):
    B, nl, _ = cl.shape
    nh = ch.shape[1]
    fl = _lin(f_low, p['low'])
    d = fl.shape[1]
    idx8 = jnp.concatenate(
        [idx3, jnp.broadcast_to(idx3[..., :1], (B, nh, K_UP_PAD - K_UP))],
        axis=2)
    table = jnp.concatenate(
        [fl, jnp.pad(cl.reshape(B * nl, 3), ((0, 0), (0, 13)))], axis=1)
    g = _gather_rows(table, idx8.reshape(-1))
    ws, bs = p['skip']
    r = B * nh
    br = min(r, 512)
    return _rowcall(functools.partial(_up_body, d),
                    [g, ch.reshape(r, 3), f_high], [K_UP_PAD, 1, 1],
                    [ws, _b2(bs)], 1, d, r, br)


def kernel(coords, features, params):
    B, N, _ = coords.shape
    c0 = coords
    c1, c2, c3, c4 = (c0[:, :N // 4], c0[:, :N // 16],
                      c0[:, :N // 64], c0[:, :N // 256])
    n4 = N // 256
    base = jnp.arange(n4, dtype=jnp.int32)[None, None, :]
    off = jnp.arange(B, dtype=jnp.int32)[:, None, None] * n4
    idx_a4 = jnp.broadcast_to(base + off, (B, n4, n4))
    idx_a0 = _knn(c0, c0, K_ATTN)
    idx_a1 = _knn(c1, c1, K_ATTN)
    idx_a2 = _knn(c2, c2, K_ATTN)
    idx_a3 = _knn(c3, c3, K_ATTN)
    idx_d1 = idx_a0[:, :N // 4]
    idx_d2 = idx_a1[:, :N // 16]
    idx_d3 = idx_a2[:, :N // 64]
    idx_d4 = idx_a3[:, :N // 256]
    idx_u6 = _knn(c3, c4, K_UP)
    idx_u7 = _knn(c2, c3, K_UP)
    idx_u8 = _knn(c1, c2, K_UP)
    idx_u9 = _knn(c0, c1, K_UP)

    f = features.reshape(B * N, 3)
    f0 = _mlp2(f, params['mlp0'])
    f0 = _pt_block(c0, f0, params['pt0'], idx_a0)
    _, f1 = _tdown(c0, f0, params['td1'], N // 4, idx_d1)
    f1 = _pt_block(c1, f1, params['pt1'], idx_a1)
    _, f2 = _tdown(c1, f1, params['td2'], N // 16, idx_d2)
    f2 = _pt_block(c2, f2, params['pt2'], idx_a2)
    _, f3 = _tdown(c2, f2, params['td3'], N // 64, idx_d3)
    f3 = _pt_block(c3, f3, params['pt3'], idx_a3)
    _, f4 = _tdown(c3, f3, params['td4'], N // 256, idx_d4)
    f4 = _pt_block(c4, f4, params['pt4'], idx_a4)
    f5 = _mlp2(f4, params['mlp2'])
    f6 = _pt_block(c4, f5, params['pt5'], idx_a4)
    f7 = _tup(c4, f6, c3, f3, params['tu6'], idx_u6)
    f7 = _pt_block(c3, f7, params['pt6'], idx_a3)
    f8 = _tup(c3, f7, c2, f2, params['tu7'], idx_u7)
    f8 = _pt_block(c2, f8, params['pt7'], idx_a2)
    f9 = _tup(c2, f8, c1, f1, params['tu8'], idx_u8)
    f9 = _pt_block(c1, f9, params['pt8'], idx_a1)
    f10 = _tup(c1, f9, c0, f0, params['tu9'], idx_u9)
    f10 = _pt_block(c0, f10, params['pt9'], idx_a0)
    logits = _mlp2(f10, params['mlp3'], logsm=True)
    return logits.reshape(B, N, -1)
